# baseline (speedup 1.0000x reference)
import functools

import jax
import jax.numpy as jnp
from jax import lax
from jax.experimental import pallas as pl
from jax.experimental.pallas import tpu as pltpu
from jax.experimental.pallas import tpu_sc as plsc

EMB = 128
NLAYER = 5
LR = 0.01
NSTEP = 3
BLK = 1000

_HI = lax.Precision.HIGHEST


def _dot(a, b, ca, cb):
    return lax.dot_general(a, b, (((ca,), (cb,)), ((), ())),
                           precision=_HI, preferred_element_type=jnp.float32)


_NC, _NS = 2, 16
_NW = _NC * _NS
_K = 128
_PAD = 16


def _pack_idx(gather_idx, scatter_idx, N):
    E = gather_idx.shape[0]
    EW = E // _NW
    nchunk = -(-EW // _K)
    nchunk += (-nchunk) % 16
    pe = nchunk * _K - EW
    g = jnp.pad(gather_idx.reshape(_NW, EW), ((0, 0), (0, pe)))
    s = jnp.pad(scatter_idx.reshape(_NW, EW), ((0, 0), (0, pe)),
                constant_values=N)
    return g.reshape(_NW, nchunk, _K), s.reshape(_NW, nchunk, _K)


@functools.lru_cache(maxsize=None)
def _make_sc_spmm(N, NCHUNK):
    HC = NCHUNK // 2
    RPS = (N // _NS) & ~7
    RLAST = N + _PAD - (_NS - 1) * RPS
    mesh = plsc.VectorSubcoreMesh(core_axis_name="c", subcore_axis_name="s")

    @functools.partial(
        pl.kernel,
        mesh=mesh,
        out_type=jax.ShapeDtypeStruct((_NC, N, EMB), jnp.float32),
        scratch_types=[
            pltpu.VMEM((HC, _K), jnp.int32),
            pltpu.VMEM((HC, _K), jnp.int32),
            pltpu.VMEM((_K, EMB), jnp.float32),
            pltpu.VMEM_SHARED((N + _PAD, EMB), jnp.float32),
            pltpu.SemaphoreType.DMA,
        ],
    )
    def spmm(h_hbm, srcp_hbm, dstp_hbm, z_hbm, out_hbm,
             src_v, dst_v, rows_v, acc, gsem):
        c = lax.axis_index("c")
        s = lax.axis_index("s")
        wid = s * _NC + c
        base = s * RPS

        @pl.when(s < _NS - 1)
        def _zero_main():
            pltpu.sync_copy(z_hbm.at[pl.ds(base, RPS)],
                            acc.at[pl.ds(base, RPS)])

        @pl.when(s == _NS - 1)
        def _zero_last():
            pltpu.sync_copy(z_hbm.at[pl.ds(base, RLAST)],
                            acc.at[pl.ds(base, RLAST)])

        plsc.subcore_barrier()

        def run_half(h0):
            pltpu.sync_copy(srcp_hbm.at[wid, pl.ds(h0 * HC, HC)], src_v)
            pltpu.sync_copy(dstp_hbm.at[wid, pl.ds(h0 * HC, HC)], dst_v)

            def body(i, carry):
                pltpu.async_copy(h_hbm.at[src_v.at[i]], rows_v, gsem).wait()
                pltpu.sync_copy(rows_v, acc.at[dst_v.at[i]], add=True)
                return carry

            lax.fori_loop(0, HC, body, 0)

        run_half(0)
        run_half(1)
        plsc.subcore_barrier()

        @pl.when(s < _NS - 1)
        def _out_main():
            pltpu.sync_copy(acc.at[pl.ds(base, RPS)],
                            out_hbm.at[c, pl.ds(base, RPS)])

        @pl.when(s == _NS - 1)
        def _out_last():
            pltpu.sync_copy(acc.at[pl.ds(base, RLAST - _PAD)],
                            out_hbm.at[c, pl.ds(base, RLAST - _PAD)])

    return spmm


def _sc_spmm(h, srcp, dstp):
    N = h.shape[0]
    z = jnp.zeros((N + _PAD, EMB), jnp.float32)
    return _make_sc_spmm(N, srcp.shape[1])(h, srcp, dstp, z)


@functools.lru_cache(maxsize=None)
def _make_fwd(N, last):
    NB = N // BLK

    def body(p0_ref, p1_ref, h_ref, w1_ref, b1_ref, w2_ref, b2_ref,
             agg_ref, r_ref, hn_ref):
        agg = p0_ref[0] + p1_ref[0] + h_ref[...]
        agg_ref[...] = agg
        z1 = _dot(agg, w1_ref[...], 1, 0) + b1_ref[...]
        r = jnp.maximum(z1, 0.0)
        r_ref[...] = r
        z2 = _dot(r, w2_ref[...], 1, 0) + b2_ref[...]
        hn_ref[...] = z2 if last else jnp.maximum(z2, 0.0)

    blk = pl.BlockSpec((BLK, EMB), lambda i: (i, 0))
    return pl.pallas_call(
        body,
        grid=(NB,),
        in_specs=[
            pl.BlockSpec((1, BLK, EMB), lambda i: (0, i, 0)),
            pl.BlockSpec((1, BLK, EMB), lambda i: (1, i, 0)),
            blk,
            pl.BlockSpec((EMB, EMB), lambda i: (0, 0)),
            pl.BlockSpec((1, EMB), lambda i: (0, 0)),
            pl.BlockSpec((EMB, EMB), lambda i: (0, 0)),
            pl.BlockSpec((1, EMB), lambda i: (0, 0)),
        ],
        out_specs=[blk, blk, blk],
        out_shape=[jax.ShapeDtypeStruct((N, EMB), jnp.float32)] * 3,
    )


@functools.lru_cache(maxsize=None)
def _make_bwd(N, last, combine, need_dagg):
    NB = N // BLK

    def body(*refs):
        refs = list(refs)
        if combine:
            q0_ref, q1_ref, dp_ref = refs[:3]
            refs = refs[3:]
            dh = q0_ref[0] + q1_ref[0] + dp_ref[...]
        else:
            dh = refs.pop(0)[...]
        if not last:
            hn_ref = refs.pop(0)
            dh = dh * (hn_ref[...] > 0).astype(jnp.float32)
        (r_ref, agg_ref, w1_ref, b1_ref, w2_ref, b2_ref) = refs[:6]
        outs = refs[6:]
        if need_dagg:
            dagg_ref = outs.pop(0)
        w1n_ref, b1n_ref, w2n_ref, b2n_ref, aW1, ab1, aW2, ab2 = outs
        i = pl.program_id(0)

        @pl.when(i == 0)
        def _init():
            aW1[...] = jnp.zeros((EMB, EMB), jnp.float32)
            ab1[...] = jnp.zeros((1, EMB), jnp.float32)
            aW2[...] = jnp.zeros((EMB, EMB), jnp.float32)
            ab2[...] = jnp.zeros((1, EMB), jnp.float32)

        r = r_ref[...]
        aW2[...] += _dot(r, dh, 0, 0)
        ab2[...] += jnp.sum(dh, axis=0, keepdims=True)
        dr = _dot(dh, w2_ref[...], 1, 1)
        dz1 = dr * (r > 0).astype(jnp.float32)
        aW1[...] += _dot(agg_ref[...], dz1, 0, 0)
        ab1[...] += jnp.sum(dz1, axis=0, keepdims=True)
        if need_dagg:
            dagg_ref[...] = _dot(dz1, w1_ref[...], 1, 1)

        @pl.when(i == NB - 1)
        def _finish():
            w1n_ref[...] = w1_ref[...] - LR * aW1[...]
            b1n_ref[...] = b1_ref[...] - LR * ab1[...]
            w2n_ref[...] = w2_ref[...] - LR * aW2[...]
            b2n_ref[...] = b2_ref[...] - LR * ab2[...]

    blk = pl.BlockSpec((BLK, EMB), lambda i: (i, 0))
    wspec = pl.BlockSpec((EMB, EMB), lambda i: (0, 0))
    bspec = pl.BlockSpec((1, EMB), lambda i: (0, 0))
    in_specs = []
    if combine:
        in_specs += [pl.BlockSpec((1, BLK, EMB), lambda i: (0, i, 0)),
                     pl.BlockSpec((1, BLK, EMB), lambda i: (1, i, 0)),
                     blk]
    else:
        in_specs += [blk]
    if not last:
        in_specs += [blk]
    in_specs += [blk, blk, wspec, bspec, wspec, bspec]
    out_specs = []
    out_shape = []
    if need_dagg:
        out_specs += [blk]
        out_shape += [jax.ShapeDtypeStruct((N, EMB), jnp.float32)]
    out_specs += [wspec, bspec, wspec, bspec]
    out_shape += [jax.ShapeDtypeStruct((EMB, EMB), jnp.float32),
                  jax.ShapeDtypeStruct((1, EMB), jnp.float32),
                  jax.ShapeDtypeStruct((EMB, EMB), jnp.float32),
                  jax.ShapeDtypeStruct((1, EMB), jnp.float32)]
    return pl.pallas_call(
        body,
        grid=(NB,),
        in_specs=in_specs,
        out_specs=out_specs,
        out_shape=out_shape,
        scratch_shapes=[pltpu.VMEM((EMB, EMB), jnp.float32),
                        pltpu.VMEM((1, EMB), jnp.float32),
                        pltpu.VMEM((EMB, EMB), jnp.float32),
                        pltpu.VMEM((1, EMB), jnp.float32)],
    )


@functools.lru_cache(maxsize=None)
def _make_head(N):
    NB = N // BLK

    def body(h_ref, b_ref, y_ref, wgt_ref, bg_ref,
             loss_ref, ds_ref, wgtn_ref, bgn_ref, sums, cnts):
        i = pl.program_id(0)

        @pl.when(i == 0)
        def _init():
            sums[...] = jnp.zeros((EMB, EMB), jnp.float32)
            cnts[...] = jnp.zeros((EMB, EMB), jnp.float32)

        bids = b_ref[0, 0]
        lane = lax.broadcasted_iota(jnp.int32, (BLK, EMB), 1)
        oh = (lane == bids[:, None]).astype(jnp.float32)
        sums[...] += _dot(oh, h_ref[...], 0, 0)
        cnts[...] += _dot(oh, jnp.ones((BLK, EMB), jnp.float32), 0, 0)

        @pl.when(i == NB - 1)
        def _finish():
            cm = jnp.maximum(cnts[...], 1.0)
            pooled = sums[...] / cm
            wgt = wgt_ref[...]
            pred = jnp.sum(pooled * wgt, axis=1, keepdims=True) + bg_ref[...]
            y = y_ref[...]
            t = (y + 1.0) * 0.5
            valid = (y * y > 1e-5).astype(jnp.float32)
            lm = (jnp.maximum(pred, 0.0) - pred * t
                  + jnp.log1p(jnp.exp(-jnp.abs(pred))))
            vs = jnp.sum(valid)
            loss_ref[...] = jnp.reshape(jnp.sum(lm * valid) / vs, (1, 1))
            dpred = (jax.nn.sigmoid(pred) - t) * valid / vs
            ds_ref[...] = dpred * wgt / cm
            wgtn_ref[...] = wgt - LR * jnp.sum(pooled * dpred, axis=0,
                                               keepdims=True)
            bgn_ref[...] = bg_ref[...] - LR * jnp.sum(dpred)

    one = pl.BlockSpec((1, 1), lambda i: (0, 0))
    emb2 = pl.BlockSpec((EMB, EMB), lambda i: (0, 0))
    return pl.pallas_call(
        body,
        grid=(NB,),
        in_specs=[
            pl.BlockSpec((BLK, EMB), lambda i: (i, 0)),
            pl.BlockSpec((1, 1, BLK), lambda i: (i, 0, 0)),
            pl.BlockSpec((EMB, 1), lambda i: (0, 0)),
            pl.BlockSpec((1, EMB), lambda i: (0, 0)),
            one,
        ],
        out_specs=[one, emb2, pl.BlockSpec((1, EMB), lambda i: (0, 0)), one],
        out_shape=[jax.ShapeDtypeStruct((1, 1), jnp.float32),
                   jax.ShapeDtypeStruct((EMB, EMB), jnp.float32),
                   jax.ShapeDtypeStruct((1, EMB), jnp.float32),
                   jax.ShapeDtypeStruct((1, 1), jnp.float32)],
        scratch_shapes=[pltpu.VMEM((EMB, EMB), jnp.float32),
                        pltpu.VMEM((EMB, EMB), jnp.float32)],
    )


@functools.lru_cache(maxsize=None)
def _make_expand(N):
    NB = N // BLK

    def body(ds_ref, b_ref, dh_ref):
        bids = b_ref[0, 0]
        lane = lax.broadcasted_iota(jnp.int32, (BLK, EMB), 1)
        oh = (lane == bids[:, None]).astype(jnp.float32)
        dh_ref[...] = _dot(oh, ds_ref[...], 1, 0)

    return pl.pallas_call(
        body,
        grid=(NB,),
        in_specs=[
            pl.BlockSpec((EMB, EMB), lambda i: (0, 0)),
            pl.BlockSpec((1, 1, BLK), lambda i: (i, 0, 0)),
        ],
        out_specs=pl.BlockSpec((BLK, EMB), lambda i: (i, 0)),
        out_shape=jax.ShapeDtypeStruct((N, EMB), jnp.float32),
    )


def _forward(x, srcp, dstp, fw, save, P0=None):
    W1s, b1s, W2s, b2s = fw[0], fw[1], fw[2], fw[3]
    N = x.shape[0]
    h = x
    aggs, rs, hs = [], [], [h]
    for l in range(NLAYER):
        P = P0 if (l == 0 and P0 is not None) else _sc_spmm(h, srcp, dstp)
        agg, r, hn = _make_fwd(N, l == NLAYER - 1)(
            P, P, h, W1s[l], b1s[l], W2s[l], b2s[l])
        if save:
            aggs.append(agg)
            rs.append(r)
            hs.append(hn)
        h = hn
    return h, aggs, rs, hs


def kernel(x_spt, edge_index_spt, batch_spt, y_spt,
           x_qry, edge_index_qry, batch_qry, y_qry,
           W1, b1, W2, b2, Wg, bg):
    N = x_spt.shape[0]
    NB = N // BLK
    src_s = edge_index_spt[0].astype(jnp.int32)
    dst_s = edge_index_spt[1].astype(jnp.int32)
    src_q = edge_index_qry[0].astype(jnp.int32)
    dst_q = edge_index_qry[1].astype(jnp.int32)
    sf_g, sf_s = _pack_idx(src_s, dst_s, N)
    st_g, st_s = _pack_idx(dst_s, src_s, N)
    qf_g, qf_s = _pack_idx(src_q, dst_q, N)
    batch_s3 = batch_spt.astype(jnp.int32).reshape(NB, 1, BLK)
    batch_q3 = batch_qry.astype(jnp.int32).reshape(NB, 1, BLK)
    y_s = jnp.pad(y_spt, (0, EMB - y_spt.shape[0])).reshape(EMB, 1)
    y_q = jnp.pad(y_qry, (0, EMB - y_qry.shape[0])).reshape(EMB, 1)

    fW1 = [W1[l] for l in range(NLAYER)]
    fb1 = [b1[l].reshape(1, EMB) for l in range(NLAYER)]
    fW2 = [W2[l] for l in range(NLAYER)]
    fb2 = [b2[l].reshape(1, EMB) for l in range(NLAYER)]
    fwgT = Wg.reshape(1, EMB)
    fbg = bg.reshape(1, 1)

    P0_spt = _sc_spmm(x_spt, sf_g, sf_s)
    for _ in range(NSTEP):
        h, aggs, rs, hs = _forward(x_spt, sf_g, sf_s,
                                   (fW1, fb1, fW2, fb2), save=True, P0=P0_spt)
        _, d_sums, fwgT_new, fbg_new = _make_head(N)(h, batch_s3, y_s, fwgT, fbg)
        dh = _make_expand(N)(d_sums, batch_s3)
        nW1 = [None] * NLAYER
        nb1 = [None] * NLAYER
        nW2 = [None] * NLAYER
        nb2 = [None] * NLAYER
        dprev = None
        Q = None
        for l in range(NLAYER - 1, -1, -1):
            last = l == NLAYER - 1
            need_dagg = l > 0
            bwd = _make_bwd(N, last, not last, need_dagg)
            args = []
            if last:
                args += [dh]
            else:
                args += [Q, Q, dprev, hs[l + 1]]
            args += [rs[l], aggs[l], fW1[l], fb1[l], fW2[l], fb2[l]]
            outs = bwd(*args)
            if need_dagg:
                dagg = outs[0]
                outs = outs[1:]
                Q = _sc_spmm(dagg, st_g, st_s)
                dprev = dagg
            nW1[l], nb1[l], nW2[l], nb2[l] = outs
        fW1, fb1, fW2, fb2 = nW1, nb1, nW2, nb2
        fwgT, fbg = fwgT_new, fbg_new

    h, _, _, _ = _forward(x_qry, qf_g, qf_s,
                          (fW1, fb1, fW2, fb2), save=False)
    loss, _, _, _ = _make_head(N)(h, batch_q3, y_q, fwgT, fbg)
    return loss[0, 0]


# speedup vs baseline: 1.0025x; 1.0025x over previous
import functools

import jax
import jax.numpy as jnp
from jax import lax
from jax.experimental import pallas as pl
from jax.experimental.pallas import tpu as pltpu
from jax.experimental.pallas import tpu_sc as plsc

EMB = 128
NLAYER = 5
LR = 0.01
NSTEP = 3
BLK = 1000

_HI = lax.Precision.HIGHEST


def _dot(a, b, ca, cb):
    return lax.dot_general(a, b, (((ca,), (cb,)), ((), ())),
                           precision=_HI, preferred_element_type=jnp.float32)


_NC, _NS = 2, 16
_NW = _NC * _NS
_K = 64
_PAD = 16


def _pack_idx(gather_idx, scatter_idx, N):
    E = gather_idx.shape[0]
    EW = E // _NW
    nchunk = -(-EW // _K)
    nchunk += (-nchunk) % 16
    pe = nchunk * _K - EW
    g = jnp.pad(gather_idx.reshape(_NW, EW), ((0, 0), (0, pe)))
    s = jnp.pad(scatter_idx.reshape(_NW, EW), ((0, 0), (0, pe)),
                constant_values=N)
    return g.reshape(_NW, nchunk, _K), s.reshape(_NW, nchunk, _K)


@functools.lru_cache(maxsize=None)
def _make_sc_spmm(N, NCHUNK):
    HC = NCHUNK // 2
    RPS = (N // _NS) & ~7
    RLAST = N + _PAD - (_NS - 1) * RPS
    mesh = plsc.VectorSubcoreMesh(core_axis_name="c", subcore_axis_name="s")

    @functools.partial(
        pl.kernel,
        mesh=mesh,
        out_type=jax.ShapeDtypeStruct((_NC, N, EMB), jnp.float32),
        scratch_types=[
            pltpu.VMEM((HC, _K), jnp.int32),
            pltpu.VMEM((HC, _K), jnp.int32),
            pltpu.VMEM((2, _K, EMB), jnp.float32),
            pltpu.VMEM_SHARED((N + _PAD, EMB), jnp.float32),
            [pltpu.SemaphoreType.DMA] * 2,
        ],
    )
    def spmm(h_hbm, srcp_hbm, dstp_hbm, z_hbm, out_hbm,
             src_v, dst_v, rows_v, acc, gsems):
        c = lax.axis_index("c")
        s = lax.axis_index("s")
        wid = s * _NC + c
        base = s * RPS

        @pl.when(s < _NS - 1)
        def _zero_main():
            pltpu.sync_copy(z_hbm.at[pl.ds(base, RPS)],
                            acc.at[pl.ds(base, RPS)])

        @pl.when(s == _NS - 1)
        def _zero_last():
            pltpu.sync_copy(z_hbm.at[pl.ds(base, RLAST)],
                            acc.at[pl.ds(base, RLAST)])

        plsc.subcore_barrier()

        def run_half(h0):
            pltpu.sync_copy(srcp_hbm.at[wid, pl.ds(h0 * HC, HC)], src_v)
            pltpu.sync_copy(dstp_hbm.at[wid, pl.ds(h0 * HC, HC)], dst_v)
            pltpu.async_copy(h_hbm.at[src_v.at[0]], rows_v.at[0], gsems[0])

            def pair(p, carry):
                i = 2 * p
                for b in range(2):
                    pltpu.make_async_copy(h_hbm.at[src_v.at[i + b]],
                                          rows_v.at[b], gsems[b]).wait()

                    @pl.when(i + b + 1 < HC)
                    def _look():
                        pltpu.async_copy(h_hbm.at[src_v.at[i + b + 1]],
                                         rows_v.at[1 - b], gsems[1 - b])

                    pltpu.sync_copy(rows_v.at[b], acc.at[dst_v.at[i + b]],
                                    add=True)
                return carry

            lax.fori_loop(0, HC // 2, pair, 0)

        run_half(0)
        run_half(1)
        plsc.subcore_barrier()

        @pl.when(s < _NS - 1)
        def _out_main():
            pltpu.sync_copy(acc.at[pl.ds(base, RPS)],
                            out_hbm.at[c, pl.ds(base, RPS)])

        @pl.when(s == _NS - 1)
        def _out_last():
            pltpu.sync_copy(acc.at[pl.ds(base, RLAST - _PAD)],
                            out_hbm.at[c, pl.ds(base, RLAST - _PAD)])

    return spmm


def _sc_spmm(h, srcp, dstp):
    N = h.shape[0]
    z = jnp.zeros((N + _PAD, EMB), jnp.float32)
    return _make_sc_spmm(N, srcp.shape[1])(h, srcp, dstp, z)


@functools.lru_cache(maxsize=None)
def _make_fwd(N, last):
    NB = N // BLK

    def body(p0_ref, p1_ref, h_ref, w1_ref, b1_ref, w2_ref, b2_ref,
             agg_ref, r_ref, hn_ref):
        agg = p0_ref[0] + p1_ref[0] + h_ref[...]
        agg_ref[...] = agg
        z1 = _dot(agg, w1_ref[...], 1, 0) + b1_ref[...]
        r = jnp.maximum(z1, 0.0)
        r_ref[...] = r
        z2 = _dot(r, w2_ref[...], 1, 0) + b2_ref[...]
        hn_ref[...] = z2 if last else jnp.maximum(z2, 0.0)

    blk = pl.BlockSpec((BLK, EMB), lambda i: (i, 0))
    return pl.pallas_call(
        body,
        grid=(NB,),
        in_specs=[
            pl.BlockSpec((1, BLK, EMB), lambda i: (0, i, 0)),
            pl.BlockSpec((1, BLK, EMB), lambda i: (1, i, 0)),
            blk,
            pl.BlockSpec((EMB, EMB), lambda i: (0, 0)),
            pl.BlockSpec((1, EMB), lambda i: (0, 0)),
            pl.BlockSpec((EMB, EMB), lambda i: (0, 0)),
            pl.BlockSpec((1, EMB), lambda i: (0, 0)),
        ],
        out_specs=[blk, blk, blk],
        out_shape=[jax.ShapeDtypeStruct((N, EMB), jnp.float32)] * 3,
    )


@functools.lru_cache(maxsize=None)
def _make_bwd(N, last, combine, need_dagg):
    NB = N // BLK

    def body(*refs):
        refs = list(refs)
        if combine:
            q0_ref, q1_ref, dp_ref = refs[:3]
            refs = refs[3:]
            dh = q0_ref[0] + q1_ref[0] + dp_ref[...]
        else:
            dh = refs.pop(0)[...]
        if not last:
            hn_ref = refs.pop(0)
            dh = dh * (hn_ref[...] > 0).astype(jnp.float32)
        (r_ref, agg_ref, w1_ref, b1_ref, w2_ref, b2_ref) = refs[:6]
        outs = refs[6:]
        if need_dagg:
            dagg_ref = outs.pop(0)
        w1n_ref, b1n_ref, w2n_ref, b2n_ref, aW1, ab1, aW2, ab2 = outs
        i = pl.program_id(0)

        @pl.when(i == 0)
        def _init():
            aW1[...] = jnp.zeros((EMB, EMB), jnp.float32)
            ab1[...] = jnp.zeros((1, EMB), jnp.float32)
            aW2[...] = jnp.zeros((EMB, EMB), jnp.float32)
            ab2[...] = jnp.zeros((1, EMB), jnp.float32)

        r = r_ref[...]
        aW2[...] += _dot(r, dh, 0, 0)
        ab2[...] += jnp.sum(dh, axis=0, keepdims=True)
        dr = _dot(dh, w2_ref[...], 1, 1)
        dz1 = dr * (r > 0).astype(jnp.float32)
        aW1[...] += _dot(agg_ref[...], dz1, 0, 0)
        ab1[...] += jnp.sum(dz1, axis=0, keepdims=True)
        if need_dagg:
            dagg_ref[...] = _dot(dz1, w1_ref[...], 1, 1)

        @pl.when(i == NB - 1)
        def _finish():
            w1n_ref[...] = w1_ref[...] - LR * aW1[...]
            b1n_ref[...] = b1_ref[...] - LR * ab1[...]
            w2n_ref[...] = w2_ref[...] - LR * aW2[...]
            b2n_ref[...] = b2_ref[...] - LR * ab2[...]

    blk = pl.BlockSpec((BLK, EMB), lambda i: (i, 0))
    wspec = pl.BlockSpec((EMB, EMB), lambda i: (0, 0))
    bspec = pl.BlockSpec((1, EMB), lambda i: (0, 0))
    in_specs = []
    if combine:
        in_specs += [pl.BlockSpec((1, BLK, EMB), lambda i: (0, i, 0)),
                     pl.BlockSpec((1, BLK, EMB), lambda i: (1, i, 0)),
                     blk]
    else:
        in_specs += [blk]
    if not last:
        in_specs += [blk]
    in_specs += [blk, blk, wspec, bspec, wspec, bspec]
    out_specs = []
    out_shape = []
    if need_dagg:
        out_specs += [blk]
        out_shape += [jax.ShapeDtypeStruct((N, EMB), jnp.float32)]
    out_specs += [wspec, bspec, wspec, bspec]
    out_shape += [jax.ShapeDtypeStruct((EMB, EMB), jnp.float32),
                  jax.ShapeDtypeStruct((1, EMB), jnp.float32),
                  jax.ShapeDtypeStruct((EMB, EMB), jnp.float32),
                  jax.ShapeDtypeStruct((1, EMB), jnp.float32)]
    return pl.pallas_call(
        body,
        grid=(NB,),
        in_specs=in_specs,
        out_specs=out_specs,
        out_shape=out_shape,
        scratch_shapes=[pltpu.VMEM((EMB, EMB), jnp.float32),
                        pltpu.VMEM((1, EMB), jnp.float32),
                        pltpu.VMEM((EMB, EMB), jnp.float32),
                        pltpu.VMEM((1, EMB), jnp.float32)],
    )


@functools.lru_cache(maxsize=None)
def _make_head(N):
    NB = N // BLK

    def body(h_ref, b_ref, y_ref, wgt_ref, bg_ref,
             loss_ref, ds_ref, wgtn_ref, bgn_ref, sums, cnts):
        i = pl.program_id(0)

        @pl.when(i == 0)
        def _init():
            sums[...] = jnp.zeros((EMB, EMB), jnp.float32)
            cnts[...] = jnp.zeros((EMB, EMB), jnp.float32)

        bids = b_ref[0, 0]
        lane = lax.broadcasted_iota(jnp.int32, (BLK, EMB), 1)
        oh = (lane == bids[:, None]).astype(jnp.float32)
        sums[...] += _dot(oh, h_ref[...], 0, 0)
        cnts[...] += _dot(oh, jnp.ones((BLK, EMB), jnp.float32), 0, 0)

        @pl.when(i == NB - 1)
        def _finish():
            cm = jnp.maximum(cnts[...], 1.0)
            pooled = sums[...] / cm
            wgt = wgt_ref[...]
            pred = jnp.sum(pooled * wgt, axis=1, keepdims=True) + bg_ref[...]
            y = y_ref[...]
            t = (y + 1.0) * 0.5
            valid = (y * y > 1e-5).astype(jnp.float32)
            lm = (jnp.maximum(pred, 0.0) - pred * t
                  + jnp.log1p(jnp.exp(-jnp.abs(pred))))
            vs = jnp.sum(valid)
            loss_ref[...] = jnp.reshape(jnp.sum(lm * valid) / vs, (1, 1))
            dpred = (jax.nn.sigmoid(pred) - t) * valid / vs
            ds_ref[...] = dpred * wgt / cm
            wgtn_ref[...] = wgt - LR * jnp.sum(pooled * dpred, axis=0,
                                               keepdims=True)
            bgn_ref[...] = bg_ref[...] - LR * jnp.sum(dpred)

    one = pl.BlockSpec((1, 1), lambda i: (0, 0))
    emb2 = pl.BlockSpec((EMB, EMB), lambda i: (0, 0))
    return pl.pallas_call(
        body,
        grid=(NB,),
        in_specs=[
            pl.BlockSpec((BLK, EMB), lambda i: (i, 0)),
            pl.BlockSpec((1, 1, BLK), lambda i: (i, 0, 0)),
            pl.BlockSpec((EMB, 1), lambda i: (0, 0)),
            pl.BlockSpec((1, EMB), lambda i: (0, 0)),
            one,
        ],
        out_specs=[one, emb2, pl.BlockSpec((1, EMB), lambda i: (0, 0)), one],
        out_shape=[jax.ShapeDtypeStruct((1, 1), jnp.float32),
                   jax.ShapeDtypeStruct((EMB, EMB), jnp.float32),
                   jax.ShapeDtypeStruct((1, EMB), jnp.float32),
                   jax.ShapeDtypeStruct((1, 1), jnp.float32)],
        scratch_shapes=[pltpu.VMEM((EMB, EMB), jnp.float32),
                        pltpu.VMEM((EMB, EMB), jnp.float32)],
    )


@functools.lru_cache(maxsize=None)
def _make_expand(N):
    NB = N // BLK

    def body(ds_ref, b_ref, dh_ref):
        bids = b_ref[0, 0]
        lane = lax.broadcasted_iota(jnp.int32, (BLK, EMB), 1)
        oh = (lane == bids[:, None]).astype(jnp.float32)
        dh_ref[...] = _dot(oh, ds_ref[...], 1, 0)

    return pl.pallas_call(
        body,
        grid=(NB,),
        in_specs=[
            pl.BlockSpec((EMB, EMB), lambda i: (0, 0)),
            pl.BlockSpec((1, 1, BLK), lambda i: (i, 0, 0)),
        ],
        out_specs=pl.BlockSpec((BLK, EMB), lambda i: (i, 0)),
        out_shape=jax.ShapeDtypeStruct((N, EMB), jnp.float32),
    )


def _forward(x, srcp, dstp, fw, save, P0=None):
    W1s, b1s, W2s, b2s = fw[0], fw[1], fw[2], fw[3]
    N = x.shape[0]
    h = x
    aggs, rs, hs = [], [], [h]
    for l in range(NLAYER):
        P = P0 if (l == 0 and P0 is not None) else _sc_spmm(h, srcp, dstp)
        agg, r, hn = _make_fwd(N, l == NLAYER - 1)(
            P, P, h, W1s[l], b1s[l], W2s[l], b2s[l])
        if save:
            aggs.append(agg)
            rs.append(r)
            hs.append(hn)
        h = hn
    return h, aggs, rs, hs


def kernel(x_spt, edge_index_spt, batch_spt, y_spt,
           x_qry, edge_index_qry, batch_qry, y_qry,
           W1, b1, W2, b2, Wg, bg):
    N = x_spt.shape[0]
    NB = N // BLK
    src_s = edge_index_spt[0].astype(jnp.int32)
    dst_s = edge_index_spt[1].astype(jnp.int32)
    src_q = edge_index_qry[0].astype(jnp.int32)
    dst_q = edge_index_qry[1].astype(jnp.int32)
    sf_g, sf_s = _pack_idx(src_s, dst_s, N)
    st_g, st_s = _pack_idx(dst_s, src_s, N)
    qf_g, qf_s = _pack_idx(src_q, dst_q, N)
    batch_s3 = batch_spt.astype(jnp.int32).reshape(NB, 1, BLK)
    batch_q3 = batch_qry.astype(jnp.int32).reshape(NB, 1, BLK)
    y_s = jnp.pad(y_spt, (0, EMB - y_spt.shape[0])).reshape(EMB, 1)
    y_q = jnp.pad(y_qry, (0, EMB - y_qry.shape[0])).reshape(EMB, 1)

    fW1 = [W1[l] for l in range(NLAYER)]
    fb1 = [b1[l].reshape(1, EMB) for l in range(NLAYER)]
    fW2 = [W2[l] for l in range(NLAYER)]
    fb2 = [b2[l].reshape(1, EMB) for l in range(NLAYER)]
    fwgT = Wg.reshape(1, EMB)
    fbg = bg.reshape(1, 1)

    P0_spt = _sc_spmm(x_spt, sf_g, sf_s)
    for _ in range(NSTEP):
        h, aggs, rs, hs = _forward(x_spt, sf_g, sf_s,
                                   (fW1, fb1, fW2, fb2), save=True, P0=P0_spt)
        _, d_sums, fwgT_new, fbg_new = _make_head(N)(h, batch_s3, y_s, fwgT, fbg)
        dh = _make_expand(N)(d_sums, batch_s3)
        nW1 = [None] * NLAYER
        nb1 = [None] * NLAYER
        nW2 = [None] * NLAYER
        nb2 = [None] * NLAYER
        dprev = None
        Q = None
        for l in range(NLAYER - 1, -1, -1):
            last = l == NLAYER - 1
            need_dagg = l > 0
            bwd = _make_bwd(N, last, not last, need_dagg)
            args = []
            if last:
                args += [dh]
            else:
                args += [Q, Q, dprev, hs[l + 1]]
            args += [rs[l], aggs[l], fW1[l], fb1[l], fW2[l], fb2[l]]
            outs = bwd(*args)
            if need_dagg:
                dagg = outs[0]
                outs = outs[1:]
                Q = _sc_spmm(dagg, st_g, st_s)
                dprev = dagg
            nW1[l], nb1[l], nW2[l], nb2[l] = outs
        fW1, fb1, fW2, fb2 = nW1, nb1, nW2, nb2
        fwgT, fbg = fwgT_new, fbg_new

    h, _, _, _ = _forward(x_qry, qf_g, qf_s,
                          (fW1, fb1, fW2, fb2), save=False)
    loss, _, _, _ = _make_head(N)(h, batch_q3, y_q, fwgT, fbg)
    return loss[0, 0]


# speedup vs baseline: 1.4226x; 1.4191x over previous
import functools

import jax
import jax.numpy as jnp
from jax import lax
from jax.experimental import pallas as pl
from jax.experimental.pallas import tpu as pltpu
from jax.experimental.pallas import tpu_sc as plsc

EMB = 128
NLAYER = 5
LR = 0.01
NSTEP = 3
BLK = 1000

_HI = lax.Precision.HIGHEST


def _dot(a, b, ca, cb):
    return lax.dot_general(a, b, (((ca,), (cb,)), ((), ())),
                           precision=_HI, preferred_element_type=jnp.float32)


_NC, _NS = 2, 16
_NW = _NC * _NS
_K = 64
_PAD = 16


def _pack_idx(gather_idx, scatter_idx, N):
    E = gather_idx.shape[0]
    EW = E // _NW
    nchunk = -(-EW // _K)
    nchunk += (-nchunk) % 16
    pe = nchunk * _K - EW
    g = jnp.pad(gather_idx.reshape(_NW, EW), ((0, 0), (0, pe)))
    s = jnp.pad(scatter_idx.reshape(_NW, EW), ((0, 0), (0, pe)),
                constant_values=N)
    return g.reshape(_NW, nchunk, _K), s.reshape(_NW, nchunk, _K)


@functools.lru_cache(maxsize=None)
def _make_sc_spmm(N, E):
    EW = E // _NW
    KE = 80
    NCHUNK = EW // KE
    RPS = (N // _NS) & ~7
    RLAST = N + _PAD - (_NS - 1) * RPS
    mesh = plsc.VectorSubcoreMesh(core_axis_name="c", subcore_axis_name="s")

    @functools.partial(
        pl.kernel,
        mesh=mesh,
        out_type=jax.ShapeDtypeStruct((_NC, N, EMB), jnp.float32),
        scratch_types=[
            pltpu.VMEM((KE,), jnp.int32),
            pltpu.VMEM((KE,), jnp.int32),
            pltpu.VMEM((KE, EMB), jnp.float32),
            pltpu.VMEM_SHARED((N + _PAD, EMB), jnp.float32),
            pltpu.SemaphoreType.DMA,
        ],
    )
    def spmm(h_hbm, src_hbm, dst_hbm, z_hbm, out_hbm,
             src_v, dst_v, rows_v, acc, gsem):
        c = lax.axis_index("c")
        s = lax.axis_index("s")
        wid = s * _NC + c
        base = s * RPS

        @pl.when(s < _NS - 1)
        def _zero_main():
            pltpu.sync_copy(z_hbm.at[pl.ds(base, RPS)],
                            acc.at[pl.ds(base, RPS)])

        @pl.when(s == _NS - 1)
        def _zero_last():
            pltpu.sync_copy(z_hbm.at[pl.ds(base, RLAST)],
                            acc.at[pl.ds(base, RLAST)])

        plsc.subcore_barrier()

        def body(i, carry):
            e0 = wid * EW + i * KE
            pltpu.sync_copy(src_hbm.at[pl.ds(e0, KE)], src_v)
            pltpu.sync_copy(dst_hbm.at[pl.ds(e0, KE)], dst_v)
            pltpu.async_copy(h_hbm.at[src_v], rows_v, gsem).wait()
            pltpu.sync_copy(rows_v, acc.at[dst_v], add=True)
            return carry

        lax.fori_loop(0, NCHUNK, body, 0)
        plsc.subcore_barrier()

        @pl.when(s < _NS - 1)
        def _out_main():
            pltpu.sync_copy(acc.at[pl.ds(base, RPS)],
                            out_hbm.at[c, pl.ds(base, RPS)])

        @pl.when(s == _NS - 1)
        def _out_last():
            pltpu.sync_copy(acc.at[pl.ds(base, RLAST - _PAD)],
                            out_hbm.at[c, pl.ds(base, RLAST - _PAD)])

    return spmm


def _sc_spmm(h, gather_idx, scatter_idx):
    N = h.shape[0]
    z = jnp.zeros((N + _PAD, EMB), jnp.float32)
    return _make_sc_spmm(N, gather_idx.shape[0])(h, gather_idx, scatter_idx, z)


@functools.lru_cache(maxsize=None)
def _make_fwd(N, last):
    NB = N // BLK

    def body(p0_ref, p1_ref, h_ref, w1_ref, b1_ref, w2_ref, b2_ref,
             agg_ref, r_ref, hn_ref):
        agg = p0_ref[0] + p1_ref[0] + h_ref[...]
        agg_ref[...] = agg
        z1 = _dot(agg, w1_ref[...], 1, 0) + b1_ref[...]
        r = jnp.maximum(z1, 0.0)
        r_ref[...] = r
        z2 = _dot(r, w2_ref[...], 1, 0) + b2_ref[...]
        hn_ref[...] = z2 if last else jnp.maximum(z2, 0.0)

    blk = pl.BlockSpec((BLK, EMB), lambda i: (i, 0))
    return pl.pallas_call(
        body,
        grid=(NB,),
        in_specs=[
            pl.BlockSpec((1, BLK, EMB), lambda i: (0, i, 0)),
            pl.BlockSpec((1, BLK, EMB), lambda i: (1, i, 0)),
            blk,
            pl.BlockSpec((EMB, EMB), lambda i: (0, 0)),
            pl.BlockSpec((1, EMB), lambda i: (0, 0)),
            pl.BlockSpec((EMB, EMB), lambda i: (0, 0)),
            pl.BlockSpec((1, EMB), lambda i: (0, 0)),
        ],
        out_specs=[blk, blk, blk],
        out_shape=[jax.ShapeDtypeStruct((N, EMB), jnp.float32)] * 3,
    )


@functools.lru_cache(maxsize=None)
def _make_bwd(N, last, combine, need_dagg):
    NB = N // BLK

    def body(*refs):
        refs = list(refs)
        if combine:
            q0_ref, q1_ref, dp_ref = refs[:3]
            refs = refs[3:]
            dh = q0_ref[0] + q1_ref[0] + dp_ref[...]
        else:
            dh = refs.pop(0)[...]
        if not last:
            hn_ref = refs.pop(0)
            dh = dh * (hn_ref[...] > 0).astype(jnp.float32)
        (r_ref, agg_ref, w1_ref, b1_ref, w2_ref, b2_ref) = refs[:6]
        outs = refs[6:]
        if need_dagg:
            dagg_ref = outs.pop(0)
        w1n_ref, b1n_ref, w2n_ref, b2n_ref, aW1, ab1, aW2, ab2 = outs
        i = pl.program_id(0)

        @pl.when(i == 0)
        def _init():
            aW1[...] = jnp.zeros((EMB, EMB), jnp.float32)
            ab1[...] = jnp.zeros((1, EMB), jnp.float32)
            aW2[...] = jnp.zeros((EMB, EMB), jnp.float32)
            ab2[...] = jnp.zeros((1, EMB), jnp.float32)

        r = r_ref[...]
        aW2[...] += _dot(r, dh, 0, 0)
        ab2[...] += jnp.sum(dh, axis=0, keepdims=True)
        dr = _dot(dh, w2_ref[...], 1, 1)
        dz1 = dr * (r > 0).astype(jnp.float32)
        aW1[...] += _dot(agg_ref[...], dz1, 0, 0)
        ab1[...] += jnp.sum(dz1, axis=0, keepdims=True)
        if need_dagg:
            dagg_ref[...] = _dot(dz1, w1_ref[...], 1, 1)

        @pl.when(i == NB - 1)
        def _finish():
            w1n_ref[...] = w1_ref[...] - LR * aW1[...]
            b1n_ref[...] = b1_ref[...] - LR * ab1[...]
            w2n_ref[...] = w2_ref[...] - LR * aW2[...]
            b2n_ref[...] = b2_ref[...] - LR * ab2[...]

    blk = pl.BlockSpec((BLK, EMB), lambda i: (i, 0))
    wspec = pl.BlockSpec((EMB, EMB), lambda i: (0, 0))
    bspec = pl.BlockSpec((1, EMB), lambda i: (0, 0))
    in_specs = []
    if combine:
        in_specs += [pl.BlockSpec((1, BLK, EMB), lambda i: (0, i, 0)),
                     pl.BlockSpec((1, BLK, EMB), lambda i: (1, i, 0)),
                     blk]
    else:
        in_specs += [blk]
    if not last:
        in_specs += [blk]
    in_specs += [blk, blk, wspec, bspec, wspec, bspec]
    out_specs = []
    out_shape = []
    if need_dagg:
        out_specs += [blk]
        out_shape += [jax.ShapeDtypeStruct((N, EMB), jnp.float32)]
    out_specs += [wspec, bspec, wspec, bspec]
    out_shape += [jax.ShapeDtypeStruct((EMB, EMB), jnp.float32),
                  jax.ShapeDtypeStruct((1, EMB), jnp.float32),
                  jax.ShapeDtypeStruct((EMB, EMB), jnp.float32),
                  jax.ShapeDtypeStruct((1, EMB), jnp.float32)]
    return pl.pallas_call(
        body,
        grid=(NB,),
        in_specs=in_specs,
        out_specs=out_specs,
        out_shape=out_shape,
        scratch_shapes=[pltpu.VMEM((EMB, EMB), jnp.float32),
                        pltpu.VMEM((1, EMB), jnp.float32),
                        pltpu.VMEM((EMB, EMB), jnp.float32),
                        pltpu.VMEM((1, EMB), jnp.float32)],
    )


@functools.lru_cache(maxsize=None)
def _make_head(N):
    NB = N // BLK

    def body(h_ref, b_ref, y_ref, wgt_ref, bg_ref,
             loss_ref, ds_ref, wgtn_ref, bgn_ref, sums, cnts):
        i = pl.program_id(0)

        @pl.when(i == 0)
        def _init():
            sums[...] = jnp.zeros((EMB, EMB), jnp.float32)
            cnts[...] = jnp.zeros((EMB, EMB), jnp.float32)

        bids = b_ref[0, 0]
        lane = lax.broadcasted_iota(jnp.int32, (BLK, EMB), 1)
        oh = (lane == bids[:, None]).astype(jnp.float32)
        sums[...] += _dot(oh, h_ref[...], 0, 0)
        cnts[...] += _dot(oh, jnp.ones((BLK, EMB), jnp.float32), 0, 0)

        @pl.when(i == NB - 1)
        def _finish():
            cm = jnp.maximum(cnts[...], 1.0)
            pooled = sums[...] / cm
            wgt = wgt_ref[...]
            pred = jnp.sum(pooled * wgt, axis=1, keepdims=True) + bg_ref[...]
            y = y_ref[...]
            t = (y + 1.0) * 0.5
            valid = (y * y > 1e-5).astype(jnp.float32)
            lm = (jnp.maximum(pred, 0.0) - pred * t
                  + jnp.log1p(jnp.exp(-jnp.abs(pred))))
            vs = jnp.sum(valid)
            loss_ref[...] = jnp.reshape(jnp.sum(lm * valid) / vs, (1, 1))
            dpred = (jax.nn.sigmoid(pred) - t) * valid / vs
            ds_ref[...] = dpred * wgt / cm
            wgtn_ref[...] = wgt - LR * jnp.sum(pooled * dpred, axis=0,
                                               keepdims=True)
            bgn_ref[...] = bg_ref[...] - LR * jnp.sum(dpred)

    one = pl.BlockSpec((1, 1), lambda i: (0, 0))
    emb2 = pl.BlockSpec((EMB, EMB), lambda i: (0, 0))
    return pl.pallas_call(
        body,
        grid=(NB,),
        in_specs=[
            pl.BlockSpec((BLK, EMB), lambda i: (i, 0)),
            pl.BlockSpec((1, 1, BLK), lambda i: (i, 0, 0)),
            pl.BlockSpec((EMB, 1), lambda i: (0, 0)),
            pl.BlockSpec((1, EMB), lambda i: (0, 0)),
            one,
        ],
        out_specs=[one, emb2, pl.BlockSpec((1, EMB), lambda i: (0, 0)), one],
        out_shape=[jax.ShapeDtypeStruct((1, 1), jnp.float32),
                   jax.ShapeDtypeStruct((EMB, EMB), jnp.float32),
                   jax.ShapeDtypeStruct((1, EMB), jnp.float32),
                   jax.ShapeDtypeStruct((1, 1), jnp.float32)],
        scratch_shapes=[pltpu.VMEM((EMB, EMB), jnp.float32),
                        pltpu.VMEM((EMB, EMB), jnp.float32)],
    )


@functools.lru_cache(maxsize=None)
def _make_expand(N):
    NB = N // BLK

    def body(ds_ref, b_ref, dh_ref):
        bids = b_ref[0, 0]
        lane = lax.broadcasted_iota(jnp.int32, (BLK, EMB), 1)
        oh = (lane == bids[:, None]).astype(jnp.float32)
        dh_ref[...] = _dot(oh, ds_ref[...], 1, 0)

    return pl.pallas_call(
        body,
        grid=(NB,),
        in_specs=[
            pl.BlockSpec((EMB, EMB), lambda i: (0, 0)),
            pl.BlockSpec((1, 1, BLK), lambda i: (i, 0, 0)),
        ],
        out_specs=pl.BlockSpec((BLK, EMB), lambda i: (i, 0)),
        out_shape=jax.ShapeDtypeStruct((N, EMB), jnp.float32),
    )


def _forward(x, srcp, dstp, fw, save, P0=None):
    W1s, b1s, W2s, b2s = fw[0], fw[1], fw[2], fw[3]
    N = x.shape[0]
    h = x
    aggs, rs, hs = [], [], [h]
    for l in range(NLAYER):
        P = P0 if (l == 0 and P0 is not None) else _sc_spmm(h, srcp, dstp)
        agg, r, hn = _make_fwd(N, l == NLAYER - 1)(
            P, P, h, W1s[l], b1s[l], W2s[l], b2s[l])
        if save:
            aggs.append(agg)
            rs.append(r)
            hs.append(hn)
        h = hn
    return h, aggs, rs, hs


def kernel(x_spt, edge_index_spt, batch_spt, y_spt,
           x_qry, edge_index_qry, batch_qry, y_qry,
           W1, b1, W2, b2, Wg, bg):
    N = x_spt.shape[0]
    NB = N // BLK
    src_s = edge_index_spt[0].astype(jnp.int32)
    dst_s = edge_index_spt[1].astype(jnp.int32)
    src_q = edge_index_qry[0].astype(jnp.int32)
    dst_q = edge_index_qry[1].astype(jnp.int32)
    sf_g, sf_s = src_s, dst_s
    st_g, st_s = dst_s, src_s
    qf_g, qf_s = src_q, dst_q
    batch_s3 = batch_spt.astype(jnp.int32).reshape(NB, 1, BLK)
    batch_q3 = batch_qry.astype(jnp.int32).reshape(NB, 1, BLK)
    y_s = jnp.pad(y_spt, (0, EMB - y_spt.shape[0])).reshape(EMB, 1)
    y_q = jnp.pad(y_qry, (0, EMB - y_qry.shape[0])).reshape(EMB, 1)

    fW1 = [W1[l] for l in range(NLAYER)]
    fb1 = [b1[l].reshape(1, EMB) for l in range(NLAYER)]
    fW2 = [W2[l] for l in range(NLAYER)]
    fb2 = [b2[l].reshape(1, EMB) for l in range(NLAYER)]
    fwgT = Wg.reshape(1, EMB)
    fbg = bg.reshape(1, 1)

    P0_spt = _sc_spmm(x_spt, sf_g, sf_s)
    for _ in range(NSTEP):
        h, aggs, rs, hs = _forward(x_spt, sf_g, sf_s,
                                   (fW1, fb1, fW2, fb2), save=True, P0=P0_spt)
        _, d_sums, fwgT_new, fbg_new = _make_head(N)(h, batch_s3, y_s, fwgT, fbg)
        dh = _make_expand(N)(d_sums, batch_s3)
        nW1 = [None] * NLAYER
        nb1 = [None] * NLAYER
        nW2 = [None] * NLAYER
        nb2 = [None] * NLAYER
        dprev = None
        Q = None
        for l in range(NLAYER - 1, -1, -1):
            last = l == NLAYER - 1
            need_dagg = l > 0
            bwd = _make_bwd(N, last, not last, need_dagg)
            args = []
            if last:
                args += [dh]
            else:
                args += [Q, Q, dprev, hs[l + 1]]
            args += [rs[l], aggs[l], fW1[l], fb1[l], fW2[l], fb2[l]]
            outs = bwd(*args)
            if need_dagg:
                dagg = outs[0]
                outs = outs[1:]
                Q = _sc_spmm(dagg, st_g, st_s)
                dprev = dagg
            nW1[l], nb1[l], nW2[l], nb2[l] = outs
        fW1, fb1, fW2, fb2 = nW1, nb1, nW2, nb2
        fwgT, fbg = fwgT_new, fbg_new

    h, _, _, _ = _forward(x_qry, qf_g, qf_s,
                          (fW1, fb1, fW2, fb2), save=False)
    loss, _, _, _ = _make_head(N)(h, batch_q3, y_q, fwgT, fbg)
    return loss[0, 0]


# speedup vs baseline: 2.9731x; 2.0899x over previous
import functools

import jax
import jax.numpy as jnp
from jax import lax
from jax.experimental import pallas as pl
from jax.experimental.pallas import tpu as pltpu
from jax.experimental.pallas import tpu_sc as plsc

EMB = 128
NLAYER = 5
LR = 0.01
NSTEP = 3
BLK = 1000

_HI = lax.Precision.HIGHEST


def _dot(a, b, ca, cb):
    return lax.dot_general(a, b, (((ca,), (cb,)), ((), ())),
                           precision=_HI, preferred_element_type=jnp.float32)


_NC, _NS = 2, 16
_NW = _NC * _NS
_K = 64
_PAD = 16


def _pack_idx(gather_idx, scatter_idx, N):
    E = gather_idx.shape[0]
    EW = E // _NW
    nchunk = -(-EW // _K)
    nchunk += (-nchunk) % 16
    pe = nchunk * _K - EW
    g = jnp.pad(gather_idx.reshape(_NW, EW), ((0, 0), (0, pe)))
    s = jnp.pad(scatter_idx.reshape(_NW, EW), ((0, 0), (0, pe)),
                constant_values=N)
    return g.reshape(_NW, nchunk, _K), s.reshape(_NW, nchunk, _K)


@functools.lru_cache(maxsize=None)
def _make_sc_spmm(N, E):
    EW = E // _NW
    KE = 80
    NCHUNK = EW // KE
    RPS = (N // _NS) & ~7
    RLAST = N + _PAD - (_NS - 1) * RPS
    mesh = plsc.VectorSubcoreMesh(core_axis_name="c", subcore_axis_name="s")

    @functools.partial(
        pl.kernel,
        mesh=mesh,
        out_type=jax.ShapeDtypeStruct((_NC, N, EMB), jnp.float32),
        scratch_types=[
            pltpu.VMEM((2, KE), jnp.int32),
            pltpu.VMEM((2, KE), jnp.int32),
            pltpu.VMEM((2, KE, EMB), jnp.float32),
            pltpu.VMEM_SHARED((N + _PAD, EMB), jnp.float32),
            [pltpu.SemaphoreType.DMA] * 2,
            [pltpu.SemaphoreType.DMA] * 4,
        ],
    )
    def spmm(h_hbm, src_hbm, dst_hbm, z_hbm, out_hbm,
             src_v, dst_v, rows_v, acc, gsems, isems):
        c = lax.axis_index("c")
        s = lax.axis_index("s")
        wid = s * _NC + c
        base = s * RPS

        @pl.when(s < _NS - 1)
        def _zero_main():
            pltpu.sync_copy(z_hbm.at[pl.ds(base, RPS)],
                            acc.at[pl.ds(base, RPS)])

        @pl.when(s == _NS - 1)
        def _zero_last():
            pltpu.sync_copy(z_hbm.at[pl.ds(base, RLAST)],
                            acc.at[pl.ds(base, RLAST)])

        plsc.subcore_barrier()

        def idx_start(b, i):
            e0 = wid * EW + i * KE
            pltpu.async_copy(src_hbm.at[pl.ds(e0, KE)], src_v.at[b],
                             isems[2 * b])
            pltpu.async_copy(dst_hbm.at[pl.ds(e0, KE)], dst_v.at[b],
                             isems[2 * b + 1])

        def idx_wait(b, i):
            e0 = wid * EW + i * KE
            pltpu.make_async_copy(src_hbm.at[pl.ds(e0, KE)], src_v.at[b],
                                  isems[2 * b]).wait()
            pltpu.make_async_copy(dst_hbm.at[pl.ds(e0, KE)], dst_v.at[b],
                                  isems[2 * b + 1]).wait()

        def gather_start(b):
            pltpu.async_copy(h_hbm.at[src_v.at[b]], rows_v.at[b], gsems[b])

        def gather_wait(b):
            pltpu.make_async_copy(h_hbm.at[src_v.at[b]], rows_v.at[b],
                                  gsems[b]).wait()

        def scatter(b):
            pltpu.sync_copy(rows_v.at[b], acc.at[dst_v.at[b]], add=True)

        idx_start(0, 0)
        idx_start(1, 1)
        idx_wait(0, 0)
        gather_start(0)
        NP = (NCHUNK - 1) // 2

        def pair(p, carry):
            i = 2 * p
            idx_wait(1, i + 1)
            gather_start(1)
            gather_wait(0)

            @pl.when(i + 2 < NCHUNK)
            def _refill0():
                idx_start(0, i + 2)

            scatter(0)

            @pl.when(i + 2 < NCHUNK)
            def _regather0():
                idx_wait(0, i + 2)
                gather_start(0)

            gather_wait(1)

            @pl.when(i + 3 < NCHUNK)
            def _refill1():
                idx_start(1, i + 3)

            scatter(1)
            return carry

        lax.fori_loop(0, NP, pair, 0)
        if NCHUNK % 2 == 1:
            gather_wait(0)
            scatter(0)
        plsc.subcore_barrier()

        @pl.when(s < _NS - 1)
        def _out_main():
            pltpu.sync_copy(acc.at[pl.ds(base, RPS)],
                            out_hbm.at[c, pl.ds(base, RPS)])

        @pl.when(s == _NS - 1)
        def _out_last():
            pltpu.sync_copy(acc.at[pl.ds(base, RLAST - _PAD)],
                            out_hbm.at[c, pl.ds(base, RLAST - _PAD)])

    return spmm


def _sc_spmm(h, gather_idx, scatter_idx):
    N = h.shape[0]
    z = jnp.zeros((N + _PAD, EMB), jnp.float32)
    return _make_sc_spmm(N, gather_idx.shape[0])(h, gather_idx, scatter_idx, z)


@functools.lru_cache(maxsize=None)
def _make_fwd(N, last):
    NB = N // BLK

    def body(p0_ref, p1_ref, h_ref, w1_ref, b1_ref, w2_ref, b2_ref,
             agg_ref, r_ref, hn_ref):
        agg = p0_ref[0] + p1_ref[0] + h_ref[...]
        agg_ref[...] = agg
        z1 = _dot(agg, w1_ref[...], 1, 0) + b1_ref[...]
        r = jnp.maximum(z1, 0.0)
        r_ref[...] = r
        z2 = _dot(r, w2_ref[...], 1, 0) + b2_ref[...]
        hn_ref[...] = z2 if last else jnp.maximum(z2, 0.0)

    blk = pl.BlockSpec((BLK, EMB), lambda i: (i, 0))
    return pl.pallas_call(
        body,
        grid=(NB,),
        in_specs=[
            pl.BlockSpec((1, BLK, EMB), lambda i: (0, i, 0)),
            pl.BlockSpec((1, BLK, EMB), lambda i: (1, i, 0)),
            blk,
            pl.BlockSpec((EMB, EMB), lambda i: (0, 0)),
            pl.BlockSpec((1, EMB), lambda i: (0, 0)),
            pl.BlockSpec((EMB, EMB), lambda i: (0, 0)),
            pl.BlockSpec((1, EMB), lambda i: (0, 0)),
        ],
        out_specs=[blk, blk, blk],
        out_shape=[jax.ShapeDtypeStruct((N, EMB), jnp.float32)] * 3,
    )


@functools.lru_cache(maxsize=None)
def _make_bwd(N, last, combine, need_dagg):
    NB = N // BLK

    def body(*refs):
        refs = list(refs)
        if combine:
            q0_ref, q1_ref, dp_ref = refs[:3]
            refs = refs[3:]
            dh = q0_ref[0] + q1_ref[0] + dp_ref[...]
        else:
            dh = refs.pop(0)[...]
        if not last:
            hn_ref = refs.pop(0)
            dh = dh * (hn_ref[...] > 0).astype(jnp.float32)
        (r_ref, agg_ref, w1_ref, b1_ref, w2_ref, b2_ref) = refs[:6]
        outs = refs[6:]
        if need_dagg:
            dagg_ref = outs.pop(0)
        w1n_ref, b1n_ref, w2n_ref, b2n_ref, aW1, ab1, aW2, ab2 = outs
        i = pl.program_id(0)

        @pl.when(i == 0)
        def _init():
            aW1[...] = jnp.zeros((EMB, EMB), jnp.float32)
            ab1[...] = jnp.zeros((1, EMB), jnp.float32)
            aW2[...] = jnp.zeros((EMB, EMB), jnp.float32)
            ab2[...] = jnp.zeros((1, EMB), jnp.float32)

        r = r_ref[...]
        aW2[...] += _dot(r, dh, 0, 0)
        ab2[...] += jnp.sum(dh, axis=0, keepdims=True)
        dr = _dot(dh, w2_ref[...], 1, 1)
        dz1 = dr * (r > 0).astype(jnp.float32)
        aW1[...] += _dot(agg_ref[...], dz1, 0, 0)
        ab1[...] += jnp.sum(dz1, axis=0, keepdims=True)
        if need_dagg:
            dagg_ref[...] = _dot(dz1, w1_ref[...], 1, 1)

        @pl.when(i == NB - 1)
        def _finish():
            w1n_ref[...] = w1_ref[...] - LR * aW1[...]
            b1n_ref[...] = b1_ref[...] - LR * ab1[...]
            w2n_ref[...] = w2_ref[...] - LR * aW2[...]
            b2n_ref[...] = b2_ref[...] - LR * ab2[...]

    blk = pl.BlockSpec((BLK, EMB), lambda i: (i, 0))
    wspec = pl.BlockSpec((EMB, EMB), lambda i: (0, 0))
    bspec = pl.BlockSpec((1, EMB), lambda i: (0, 0))
    in_specs = []
    if combine:
        in_specs += [pl.BlockSpec((1, BLK, EMB), lambda i: (0, i, 0)),
                     pl.BlockSpec((1, BLK, EMB), lambda i: (1, i, 0)),
                     blk]
    else:
        in_specs += [blk]
    if not last:
        in_specs += [blk]
    in_specs += [blk, blk, wspec, bspec, wspec, bspec]
    out_specs = []
    out_shape = []
    if need_dagg:
        out_specs += [blk]
        out_shape += [jax.ShapeDtypeStruct((N, EMB), jnp.float32)]
    out_specs += [wspec, bspec, wspec, bspec]
    out_shape += [jax.ShapeDtypeStruct((EMB, EMB), jnp.float32),
                  jax.ShapeDtypeStruct((1, EMB), jnp.float32),
                  jax.ShapeDtypeStruct((EMB, EMB), jnp.float32),
                  jax.ShapeDtypeStruct((1, EMB), jnp.float32)]
    return pl.pallas_call(
        body,
        grid=(NB,),
        in_specs=in_specs,
        out_specs=out_specs,
        out_shape=out_shape,
        scratch_shapes=[pltpu.VMEM((EMB, EMB), jnp.float32),
                        pltpu.VMEM((1, EMB), jnp.float32),
                        pltpu.VMEM((EMB, EMB), jnp.float32),
                        pltpu.VMEM((1, EMB), jnp.float32)],
    )


@functools.lru_cache(maxsize=None)
def _make_head(N):
    NB = N // BLK

    def body(h_ref, b_ref, y_ref, wgt_ref, bg_ref,
             loss_ref, ds_ref, wgtn_ref, bgn_ref, sums, cnts):
        i = pl.program_id(0)

        @pl.when(i == 0)
        def _init():
            sums[...] = jnp.zeros((EMB, EMB), jnp.float32)
            cnts[...] = jnp.zeros((EMB, EMB), jnp.float32)

        bids = b_ref[0, 0]
        lane = lax.broadcasted_iota(jnp.int32, (BLK, EMB), 1)
        oh = (lane == bids[:, None]).astype(jnp.float32)
        sums[...] += _dot(oh, h_ref[...], 0, 0)
        cnts[...] += _dot(oh, jnp.ones((BLK, EMB), jnp.float32), 0, 0)

        @pl.when(i == NB - 1)
        def _finish():
            cm = jnp.maximum(cnts[...], 1.0)
            pooled = sums[...] / cm
            wgt = wgt_ref[...]
            pred = jnp.sum(pooled * wgt, axis=1, keepdims=True) + bg_ref[...]
            y = y_ref[...]
            t = (y + 1.0) * 0.5
            valid = (y * y > 1e-5).astype(jnp.float32)
            lm = (jnp.maximum(pred, 0.0) - pred * t
                  + jnp.log1p(jnp.exp(-jnp.abs(pred))))
            vs = jnp.sum(valid)
            loss_ref[...] = jnp.reshape(jnp.sum(lm * valid) / vs, (1, 1))
            dpred = (jax.nn.sigmoid(pred) - t) * valid / vs
            ds_ref[...] = dpred * wgt / cm
            wgtn_ref[...] = wgt - LR * jnp.sum(pooled * dpred, axis=0,
                                               keepdims=True)
            bgn_ref[...] = bg_ref[...] - LR * jnp.sum(dpred)

    one = pl.BlockSpec((1, 1), lambda i: (0, 0))
    emb2 = pl.BlockSpec((EMB, EMB), lambda i: (0, 0))
    return pl.pallas_call(
        body,
        grid=(NB,),
        in_specs=[
            pl.BlockSpec((BLK, EMB), lambda i: (i, 0)),
            pl.BlockSpec((1, 1, BLK), lambda i: (i, 0, 0)),
            pl.BlockSpec((EMB, 1), lambda i: (0, 0)),
            pl.BlockSpec((1, EMB), lambda i: (0, 0)),
            one,
        ],
        out_specs=[one, emb2, pl.BlockSpec((1, EMB), lambda i: (0, 0)), one],
        out_shape=[jax.ShapeDtypeStruct((1, 1), jnp.float32),
                   jax.ShapeDtypeStruct((EMB, EMB), jnp.float32),
                   jax.ShapeDtypeStruct((1, EMB), jnp.float32),
                   jax.ShapeDtypeStruct((1, 1), jnp.float32)],
        scratch_shapes=[pltpu.VMEM((EMB, EMB), jnp.float32),
                        pltpu.VMEM((EMB, EMB), jnp.float32)],
    )


@functools.lru_cache(maxsize=None)
def _make_expand(N):
    NB = N // BLK

    def body(ds_ref, b_ref, dh_ref):
        bids = b_ref[0, 0]
        lane = lax.broadcasted_iota(jnp.int32, (BLK, EMB), 1)
        oh = (lane == bids[:, None]).astype(jnp.float32)
        dh_ref[...] = _dot(oh, ds_ref[...], 1, 0)

    return pl.pallas_call(
        body,
        grid=(NB,),
        in_specs=[
            pl.BlockSpec((EMB, EMB), lambda i: (0, 0)),
            pl.BlockSpec((1, 1, BLK), lambda i: (i, 0, 0)),
        ],
        out_specs=pl.BlockSpec((BLK, EMB), lambda i: (i, 0)),
        out_shape=jax.ShapeDtypeStruct((N, EMB), jnp.float32),
    )


def _forward(x, srcp, dstp, fw, save, P0=None):
    W1s, b1s, W2s, b2s = fw[0], fw[1], fw[2], fw[3]
    N = x.shape[0]
    h = x
    aggs, rs, hs = [], [], [h]
    for l in range(NLAYER):
        P = P0 if (l == 0 and P0 is not None) else _sc_spmm(h, srcp, dstp)
        agg, r, hn = _make_fwd(N, l == NLAYER - 1)(
            P, P, h, W1s[l], b1s[l], W2s[l], b2s[l])
        if save:
            aggs.append(agg)
            rs.append(r)
            hs.append(hn)
        h = hn
    return h, aggs, rs, hs


def kernel(x_spt, edge_index_spt, batch_spt, y_spt,
           x_qry, edge_index_qry, batch_qry, y_qry,
           W1, b1, W2, b2, Wg, bg):
    N = x_spt.shape[0]
    NB = N // BLK
    src_s = edge_index_spt[0].astype(jnp.int32)
    dst_s = edge_index_spt[1].astype(jnp.int32)
    src_q = edge_index_qry[0].astype(jnp.int32)
    dst_q = edge_index_qry[1].astype(jnp.int32)
    sf_g, sf_s = src_s, dst_s
    st_g, st_s = dst_s, src_s
    qf_g, qf_s = src_q, dst_q
    batch_s3 = batch_spt.astype(jnp.int32).reshape(NB, 1, BLK)
    batch_q3 = batch_qry.astype(jnp.int32).reshape(NB, 1, BLK)
    y_s = jnp.pad(y_spt, (0, EMB - y_spt.shape[0])).reshape(EMB, 1)
    y_q = jnp.pad(y_qry, (0, EMB - y_qry.shape[0])).reshape(EMB, 1)

    fW1 = [W1[l] for l in range(NLAYER)]
    fb1 = [b1[l].reshape(1, EMB) for l in range(NLAYER)]
    fW2 = [W2[l] for l in range(NLAYER)]
    fb2 = [b2[l].reshape(1, EMB) for l in range(NLAYER)]
    fwgT = Wg.reshape(1, EMB)
    fbg = bg.reshape(1, 1)

    P0_spt = _sc_spmm(x_spt, sf_g, sf_s)
    for _ in range(NSTEP):
        h, aggs, rs, hs = _forward(x_spt, sf_g, sf_s,
                                   (fW1, fb1, fW2, fb2), save=True, P0=P0_spt)
        _, d_sums, fwgT_new, fbg_new = _make_head(N)(h, batch_s3, y_s, fwgT, fbg)
        dh = _make_expand(N)(d_sums, batch_s3)
        nW1 = [None] * NLAYER
        nb1 = [None] * NLAYER
        nW2 = [None] * NLAYER
        nb2 = [None] * NLAYER
        dprev = None
        Q = None
        for l in range(NLAYER - 1, -1, -1):
            last = l == NLAYER - 1
            need_dagg = l > 0
            bwd = _make_bwd(N, last, not last, need_dagg)
            args = []
            if last:
                args += [dh]
            else:
                args += [Q, Q, dprev, hs[l + 1]]
            args += [rs[l], aggs[l], fW1[l], fb1[l], fW2[l], fb2[l]]
            outs = bwd(*args)
            if need_dagg:
                dagg = outs[0]
                outs = outs[1:]
                Q = _sc_spmm(dagg, st_g, st_s)
                dprev = dagg
            nW1[l], nb1[l], nW2[l], nb2[l] = outs
        fW1, fb1, fW2, fb2 = nW1, nb1, nW2, nb2
        fwgT, fbg = fwgT_new, fbg_new

    h, _, _, _ = _forward(x_qry, qf_g, qf_s,
                          (fW1, fb1, fW2, fb2), save=False)
    loss, _, _, _ = _make_head(N)(h, batch_q3, y_q, fwgT, fbg)
    return loss[0, 0]


# speedup vs baseline: 3.0762x; 1.0347x over previous
import functools

import jax
import jax.numpy as jnp
from jax import lax
from jax.experimental import pallas as pl
from jax.experimental.pallas import tpu as pltpu
from jax.experimental.pallas import tpu_sc as plsc

EMB = 128
NLAYER = 5
LR = 0.01
NSTEP = 3
BLK = 1000

_HI = lax.Precision.HIGHEST


def _dot(a, b, ca, cb):
    return lax.dot_general(a, b, (((ca,), (cb,)), ((), ())),
                           precision=_HI, preferred_element_type=jnp.float32)


_NC, _NS = 2, 16
_NW = _NC * _NS
_K = 64
_PAD = 16


def _pack_idx(gather_idx, scatter_idx, N):
    E = gather_idx.shape[0]
    EW = E // _NW
    nchunk = -(-EW // _K)
    nchunk += (-nchunk) % 16
    pe = nchunk * _K - EW
    g = jnp.pad(gather_idx.reshape(_NW, EW), ((0, 0), (0, pe)))
    s = jnp.pad(scatter_idx.reshape(_NW, EW), ((0, 0), (0, pe)),
                constant_values=N)
    return g.reshape(_NW, nchunk, _K), s.reshape(_NW, nchunk, _K)


@functools.lru_cache(maxsize=None)
def _make_sc_spmm(N, E):
    EW = E // _NW
    KE = 80
    NCHUNK = EW // KE
    RPS = (N // _NS) & ~7
    RLAST = N + _PAD - (_NS - 1) * RPS
    mesh = plsc.VectorSubcoreMesh(core_axis_name="c", subcore_axis_name="s")

    @functools.partial(
        pl.kernel,
        mesh=mesh,
        out_type=jax.ShapeDtypeStruct((_NC, N, EMB), jnp.float32),
        scratch_types=[
            pltpu.VMEM((3, KE), jnp.int32),
            pltpu.VMEM((3, KE), jnp.int32),
            pltpu.VMEM((3, KE, EMB), jnp.float32),
            pltpu.VMEM_SHARED((N + _PAD, EMB), jnp.float32),
            [pltpu.SemaphoreType.DMA] * 3,
            [pltpu.SemaphoreType.DMA] * 6,
        ],
    )
    def spmm(h_hbm, src_hbm, dst_hbm, z_hbm, out_hbm,
             src_v, dst_v, rows_v, acc, gsems, isems):
        c = lax.axis_index("c")
        s = lax.axis_index("s")
        wid = s * _NC + c
        base = s * RPS

        @pl.when(s < _NS - 1)
        def _zero_main():
            pltpu.sync_copy(z_hbm.at[pl.ds(base, RPS)],
                            acc.at[pl.ds(base, RPS)])

        @pl.when(s == _NS - 1)
        def _zero_last():
            pltpu.sync_copy(z_hbm.at[pl.ds(base, RLAST)],
                            acc.at[pl.ds(base, RLAST)])

        plsc.subcore_barrier()

        def idx_start(b, i):
            e0 = wid * EW + i * KE
            pltpu.async_copy(src_hbm.at[pl.ds(e0, KE)], src_v.at[b],
                             isems[2 * b])
            pltpu.async_copy(dst_hbm.at[pl.ds(e0, KE)], dst_v.at[b],
                             isems[2 * b + 1])

        def idx_wait(b, i):
            e0 = wid * EW + i * KE
            pltpu.make_async_copy(src_hbm.at[pl.ds(e0, KE)], src_v.at[b],
                                  isems[2 * b]).wait()
            pltpu.make_async_copy(dst_hbm.at[pl.ds(e0, KE)], dst_v.at[b],
                                  isems[2 * b + 1]).wait()

        def gather_start(b):
            pltpu.async_copy(h_hbm.at[src_v.at[b]], rows_v.at[b], gsems[b])

        def gather_wait(b):
            pltpu.make_async_copy(h_hbm.at[src_v.at[b]], rows_v.at[b],
                                  gsems[b]).wait()

        def scatter(b):
            pltpu.sync_copy(rows_v.at[b], acc.at[dst_v.at[b]], add=True)

        for b in range(3):
            idx_start(b, b)
        idx_wait(0, 0)
        gather_start(0)
        idx_wait(1, 1)
        gather_start(1)
        NT = NCHUNK // 3

        def rot(p, carry):
            i0 = 3 * p
            for b in range(3):
                i = i0 + b
                gather_wait(b)
                scatter(b)

                @pl.when(i + 3 < NCHUNK)
                def _refill():
                    idx_start(b, i + 3)

                b2 = (b + 2) % 3

                @pl.when(i + 2 < NCHUNK)
                def _regather():
                    idx_wait(b2, i + 2)
                    gather_start(b2)

            return carry

        lax.fori_loop(0, NT, rot, 0)
        for i in range(3 * NT, NCHUNK):
            b = i % 3
            gather_wait(b)
            scatter(b)
        plsc.subcore_barrier()

        @pl.when(s < _NS - 1)
        def _out_main():
            pltpu.sync_copy(acc.at[pl.ds(base, RPS)],
                            out_hbm.at[c, pl.ds(base, RPS)])

        @pl.when(s == _NS - 1)
        def _out_last():
            pltpu.sync_copy(acc.at[pl.ds(base, RLAST - _PAD)],
                            out_hbm.at[c, pl.ds(base, RLAST - _PAD)])

    return spmm


def _sc_spmm(h, gather_idx, scatter_idx):
    N = h.shape[0]
    z = jnp.zeros((N + _PAD, EMB), jnp.float32)
    return _make_sc_spmm(N, gather_idx.shape[0])(h, gather_idx, scatter_idx, z)


@functools.lru_cache(maxsize=None)
def _make_fwd(N, last):
    NB = N // BLK

    def body(p0_ref, p1_ref, h_ref, w1_ref, b1_ref, w2_ref, b2_ref,
             agg_ref, r_ref, hn_ref):
        agg = p0_ref[0] + p1_ref[0] + h_ref[...]
        agg_ref[...] = agg
        z1 = _dot(agg, w1_ref[...], 1, 0) + b1_ref[...]
        r = jnp.maximum(z1, 0.0)
        r_ref[...] = r
        z2 = _dot(r, w2_ref[...], 1, 0) + b2_ref[...]
        hn_ref[...] = z2 if last else jnp.maximum(z2, 0.0)

    blk = pl.BlockSpec((BLK, EMB), lambda i: (i, 0))
    return pl.pallas_call(
        body,
        grid=(NB,),
        in_specs=[
            pl.BlockSpec((1, BLK, EMB), lambda i: (0, i, 0)),
            pl.BlockSpec((1, BLK, EMB), lambda i: (1, i, 0)),
            blk,
            pl.BlockSpec((EMB, EMB), lambda i: (0, 0)),
            pl.BlockSpec((1, EMB), lambda i: (0, 0)),
            pl.BlockSpec((EMB, EMB), lambda i: (0, 0)),
            pl.BlockSpec((1, EMB), lambda i: (0, 0)),
        ],
        out_specs=[blk, blk, blk],
        out_shape=[jax.ShapeDtypeStruct((N, EMB), jnp.float32)] * 3,
    )


@functools.lru_cache(maxsize=None)
def _make_bwd(N, last, combine, need_dagg):
    NB = N // BLK

    def body(*refs):
        refs = list(refs)
        if combine:
            q0_ref, q1_ref, dp_ref = refs[:3]
            refs = refs[3:]
            dh = q0_ref[0] + q1_ref[0] + dp_ref[...]
        else:
            dh = refs.pop(0)[...]
        if not last:
            hn_ref = refs.pop(0)
            dh = dh * (hn_ref[...] > 0).astype(jnp.float32)
        (r_ref, agg_ref, w1_ref, b1_ref, w2_ref, b2_ref) = refs[:6]
        outs = refs[6:]
        if need_dagg:
            dagg_ref = outs.pop(0)
        w1n_ref, b1n_ref, w2n_ref, b2n_ref, aW1, ab1, aW2, ab2 = outs
        i = pl.program_id(0)

        @pl.when(i == 0)
        def _init():
            aW1[...] = jnp.zeros((EMB, EMB), jnp.float32)
            ab1[...] = jnp.zeros((1, EMB), jnp.float32)
            aW2[...] = jnp.zeros((EMB, EMB), jnp.float32)
            ab2[...] = jnp.zeros((1, EMB), jnp.float32)

        r = r_ref[...]
        aW2[...] += _dot(r, dh, 0, 0)
        ab2[...] += jnp.sum(dh, axis=0, keepdims=True)
        dr = _dot(dh, w2_ref[...], 1, 1)
        dz1 = dr * (r > 0).astype(jnp.float32)
        aW1[...] += _dot(agg_ref[...], dz1, 0, 0)
        ab1[...] += jnp.sum(dz1, axis=0, keepdims=True)
        if need_dagg:
            dagg_ref[...] = _dot(dz1, w1_ref[...], 1, 1)

        @pl.when(i == NB - 1)
        def _finish():
            w1n_ref[...] = w1_ref[...] - LR * aW1[...]
            b1n_ref[...] = b1_ref[...] - LR * ab1[...]
            w2n_ref[...] = w2_ref[...] - LR * aW2[...]
            b2n_ref[...] = b2_ref[...] - LR * ab2[...]

    blk = pl.BlockSpec((BLK, EMB), lambda i: (i, 0))
    wspec = pl.BlockSpec((EMB, EMB), lambda i: (0, 0))
    bspec = pl.BlockSpec((1, EMB), lambda i: (0, 0))
    in_specs = []
    if combine:
        in_specs += [pl.BlockSpec((1, BLK, EMB), lambda i: (0, i, 0)),
                     pl.BlockSpec((1, BLK, EMB), lambda i: (1, i, 0)),
                     blk]
    else:
        in_specs += [blk]
    if not last:
        in_specs += [blk]
    in_specs += [blk, blk, wspec, bspec, wspec, bspec]
    out_specs = []
    out_shape = []
    if need_dagg:
        out_specs += [blk]
        out_shape += [jax.ShapeDtypeStruct((N, EMB), jnp.float32)]
    out_specs += [wspec, bspec, wspec, bspec]
    out_shape += [jax.ShapeDtypeStruct((EMB, EMB), jnp.float32),
                  jax.ShapeDtypeStruct((1, EMB), jnp.float32),
                  jax.ShapeDtypeStruct((EMB, EMB), jnp.float32),
                  jax.ShapeDtypeStruct((1, EMB), jnp.float32)]
    return pl.pallas_call(
        body,
        grid=(NB,),
        in_specs=in_specs,
        out_specs=out_specs,
        out_shape=out_shape,
        scratch_shapes=[pltpu.VMEM((EMB, EMB), jnp.float32),
                        pltpu.VMEM((1, EMB), jnp.float32),
                        pltpu.VMEM((EMB, EMB), jnp.float32),
                        pltpu.VMEM((1, EMB), jnp.float32)],
    )


@functools.lru_cache(maxsize=None)
def _make_head(N):
    NB = N // BLK

    def body(h_ref, b_ref, y_ref, wgt_ref, bg_ref,
             loss_ref, ds_ref, wgtn_ref, bgn_ref, sums, cnts):
        i = pl.program_id(0)

        @pl.when(i == 0)
        def _init():
            sums[...] = jnp.zeros((EMB, EMB), jnp.float32)
            cnts[...] = jnp.zeros((EMB, EMB), jnp.float32)

        bids = b_ref[0, 0]
        lane = lax.broadcasted_iota(jnp.int32, (BLK, EMB), 1)
        oh = (lane == bids[:, None]).astype(jnp.float32)
        sums[...] += _dot(oh, h_ref[...], 0, 0)
        cnts[...] += _dot(oh, jnp.ones((BLK, EMB), jnp.float32), 0, 0)

        @pl.when(i == NB - 1)
        def _finish():
            cm = jnp.maximum(cnts[...], 1.0)
            pooled = sums[...] / cm
            wgt = wgt_ref[...]
            pred = jnp.sum(pooled * wgt, axis=1, keepdims=True) + bg_ref[...]
            y = y_ref[...]
            t = (y + 1.0) * 0.5
            valid = (y * y > 1e-5).astype(jnp.float32)
            lm = (jnp.maximum(pred, 0.0) - pred * t
                  + jnp.log1p(jnp.exp(-jnp.abs(pred))))
            vs = jnp.sum(valid)
            loss_ref[...] = jnp.reshape(jnp.sum(lm * valid) / vs, (1, 1))
            dpred = (jax.nn.sigmoid(pred) - t) * valid / vs
            ds_ref[...] = dpred * wgt / cm
            wgtn_ref[...] = wgt - LR * jnp.sum(pooled * dpred, axis=0,
                                               keepdims=True)
            bgn_ref[...] = bg_ref[...] - LR * jnp.sum(dpred)

    one = pl.BlockSpec((1, 1), lambda i: (0, 0))
    emb2 = pl.BlockSpec((EMB, EMB), lambda i: (0, 0))
    return pl.pallas_call(
        body,
        grid=(NB,),
        in_specs=[
            pl.BlockSpec((BLK, EMB), lambda i: (i, 0)),
            pl.BlockSpec((1, 1, BLK), lambda i: (i, 0, 0)),
            pl.BlockSpec((EMB, 1), lambda i: (0, 0)),
            pl.BlockSpec((1, EMB), lambda i: (0, 0)),
            one,
        ],
        out_specs=[one, emb2, pl.BlockSpec((1, EMB), lambda i: (0, 0)), one],
        out_shape=[jax.ShapeDtypeStruct((1, 1), jnp.float32),
                   jax.ShapeDtypeStruct((EMB, EMB), jnp.float32),
                   jax.ShapeDtypeStruct((1, EMB), jnp.float32),
                   jax.ShapeDtypeStruct((1, 1), jnp.float32)],
        scratch_shapes=[pltpu.VMEM((EMB, EMB), jnp.float32),
                        pltpu.VMEM((EMB, EMB), jnp.float32)],
    )


@functools.lru_cache(maxsize=None)
def _make_expand(N):
    NB = N // BLK

    def body(ds_ref, b_ref, dh_ref):
        bids = b_ref[0, 0]
        lane = lax.broadcasted_iota(jnp.int32, (BLK, EMB), 1)
        oh = (lane == bids[:, None]).astype(jnp.float32)
        dh_ref[...] = _dot(oh, ds_ref[...], 1, 0)

    return pl.pallas_call(
        body,
        grid=(NB,),
        in_specs=[
            pl.BlockSpec((EMB, EMB), lambda i: (0, 0)),
            pl.BlockSpec((1, 1, BLK), lambda i: (i, 0, 0)),
        ],
        out_specs=pl.BlockSpec((BLK, EMB), lambda i: (i, 0)),
        out_shape=jax.ShapeDtypeStruct((N, EMB), jnp.float32),
    )


def _forward(x, srcp, dstp, fw, save, P0=None):
    W1s, b1s, W2s, b2s = fw[0], fw[1], fw[2], fw[3]
    N = x.shape[0]
    h = x
    aggs, rs, hs = [], [], [h]
    for l in range(NLAYER):
        P = P0 if (l == 0 and P0 is not None) else _sc_spmm(h, srcp, dstp)
        agg, r, hn = _make_fwd(N, l == NLAYER - 1)(
            P, P, h, W1s[l], b1s[l], W2s[l], b2s[l])
        if save:
            aggs.append(agg)
            rs.append(r)
            hs.append(hn)
        h = hn
    return h, aggs, rs, hs


def kernel(x_spt, edge_index_spt, batch_spt, y_spt,
           x_qry, edge_index_qry, batch_qry, y_qry,
           W1, b1, W2, b2, Wg, bg):
    N = x_spt.shape[0]
    NB = N // BLK
    src_s = edge_index_spt[0].astype(jnp.int32)
    dst_s = edge_index_spt[1].astype(jnp.int32)
    src_q = edge_index_qry[0].astype(jnp.int32)
    dst_q = edge_index_qry[1].astype(jnp.int32)
    sf_g, sf_s = src_s, dst_s
    st_g, st_s = dst_s, src_s
    qf_g, qf_s = src_q, dst_q
    batch_s3 = batch_spt.astype(jnp.int32).reshape(NB, 1, BLK)
    batch_q3 = batch_qry.astype(jnp.int32).reshape(NB, 1, BLK)
    y_s = jnp.pad(y_spt, (0, EMB - y_spt.shape[0])).reshape(EMB, 1)
    y_q = jnp.pad(y_qry, (0, EMB - y_qry.shape[0])).reshape(EMB, 1)

    fW1 = [W1[l] for l in range(NLAYER)]
    fb1 = [b1[l].reshape(1, EMB) for l in range(NLAYER)]
    fW2 = [W2[l] for l in range(NLAYER)]
    fb2 = [b2[l].reshape(1, EMB) for l in range(NLAYER)]
    fwgT = Wg.reshape(1, EMB)
    fbg = bg.reshape(1, 1)

    P0_spt = _sc_spmm(x_spt, sf_g, sf_s)
    for _ in range(NSTEP):
        h, aggs, rs, hs = _forward(x_spt, sf_g, sf_s,
                                   (fW1, fb1, fW2, fb2), save=True, P0=P0_spt)
        _, d_sums, fwgT_new, fbg_new = _make_head(N)(h, batch_s3, y_s, fwgT, fbg)
        dh = _make_expand(N)(d_sums, batch_s3)
        nW1 = [None] * NLAYER
        nb1 = [None] * NLAYER
        nW2 = [None] * NLAYER
        nb2 = [None] * NLAYER
        dprev = None
        Q = None
        for l in range(NLAYER - 1, -1, -1):
            last = l == NLAYER - 1
            need_dagg = l > 0
            bwd = _make_bwd(N, last, not last, need_dagg)
            args = []
            if last:
                args += [dh]
            else:
                args += [Q, Q, dprev, hs[l + 1]]
            args += [rs[l], aggs[l], fW1[l], fb1[l], fW2[l], fb2[l]]
            outs = bwd(*args)
            if need_dagg:
                dagg = outs[0]
                outs = outs[1:]
                Q = _sc_spmm(dagg, st_g, st_s)
                dprev = dagg
            nW1[l], nb1[l], nW2[l], nb2[l] = outs
        fW1, fb1, fW2, fb2 = nW1, nb1, nW2, nb2
        fwgT, fbg = fwgT_new, fbg_new

    h, _, _, _ = _forward(x_qry, qf_g, qf_s,
                          (fW1, fb1, fW2, fb2), save=False)
    loss, _, _, _ = _make_head(N)(h, batch_q3, y_q, fwgT, fbg)
    return loss[0, 0]


# speedup vs baseline: 3.0792x; 1.0010x over previous
import functools

import jax
import jax.numpy as jnp
from jax import lax
from jax.experimental import pallas as pl
from jax.experimental.pallas import tpu as pltpu
from jax.experimental.pallas import tpu_sc as plsc

EMB = 128
NLAYER = 5
LR = 0.01
NSTEP = 3
BLK = 1000

_HI = lax.Precision.HIGHEST


def _dot(a, b, ca, cb):
    return lax.dot_general(a, b, (((ca,), (cb,)), ((), ())),
                           precision=_HI, preferred_element_type=jnp.float32)


_NC, _NS = 2, 16
_NW = _NC * _NS
_PAD = 16


@functools.lru_cache(maxsize=None)
def _make_sc_spmm(N, E):
    EW = E // _NW
    KE = 80
    NCHUNK = EW // KE
    RPS = (N // _NS) & ~7
    RLAST = N + _PAD - (_NS - 1) * RPS
    mesh = plsc.VectorSubcoreMesh(core_axis_name="c", subcore_axis_name="s")

    @functools.partial(
        pl.kernel,
        mesh=mesh,
        out_type=jax.ShapeDtypeStruct((_NC, N, EMB), jnp.float32),
        scratch_types=[
            pltpu.VMEM((3, KE), jnp.int32),
            pltpu.VMEM((3, KE), jnp.int32),
            pltpu.VMEM((3, KE, EMB), jnp.float32),
            pltpu.VMEM_SHARED((N + _PAD, EMB), jnp.float32),
            [pltpu.SemaphoreType.DMA] * 3,
            [pltpu.SemaphoreType.DMA] * 6,
        ],
    )
    def spmm(h_hbm, src_hbm, dst_hbm, z_hbm, out_hbm,
             src_v, dst_v, rows_v, acc, gsems, isems):
        c = lax.axis_index("c")
        s = lax.axis_index("s")
        wid = s * _NC + c
        base = s * RPS

        @pl.when(s < _NS - 1)
        def _zero_main():
            pltpu.sync_copy(z_hbm.at[pl.ds(base, RPS)],
                            acc.at[pl.ds(base, RPS)])

        @pl.when(s == _NS - 1)
        def _zero_last():
            pltpu.sync_copy(z_hbm.at[pl.ds(base, RLAST)],
                            acc.at[pl.ds(base, RLAST)])

        plsc.subcore_barrier()

        def idx_start(b, i):
            e0 = wid * EW + i * KE
            pltpu.async_copy(src_hbm.at[pl.ds(e0, KE)], src_v.at[b],
                             isems[2 * b])
            pltpu.async_copy(dst_hbm.at[pl.ds(e0, KE)], dst_v.at[b],
                             isems[2 * b + 1])

        def idx_wait(b, i):
            e0 = wid * EW + i * KE
            pltpu.make_async_copy(src_hbm.at[pl.ds(e0, KE)], src_v.at[b],
                                  isems[2 * b]).wait()
            pltpu.make_async_copy(dst_hbm.at[pl.ds(e0, KE)], dst_v.at[b],
                                  isems[2 * b + 1]).wait()

        def gather_start(b):
            pltpu.async_copy(h_hbm.at[src_v.at[b]], rows_v.at[b], gsems[b])

        def gather_wait(b):
            pltpu.make_async_copy(h_hbm.at[src_v.at[b]], rows_v.at[b],
                                  gsems[b]).wait()

        def scatter(b):
            pltpu.sync_copy(rows_v.at[b], acc.at[dst_v.at[b]], add=True)

        for b in range(3):
            idx_start(b, b)
        idx_wait(0, 0)
        gather_start(0)
        idx_wait(1, 1)
        gather_start(1)
        NT = NCHUNK // 3

        def rot(p, carry):
            i0 = 3 * p
            for b in range(3):
                i = i0 + b
                gather_wait(b)
                scatter(b)

                @pl.when(i + 3 < NCHUNK)
                def _refill():
                    idx_start(b, i + 3)

                b2 = (b + 2) % 3

                @pl.when(i + 2 < NCHUNK)
                def _regather():
                    idx_wait(b2, i + 2)
                    gather_start(b2)

            return carry

        lax.fori_loop(0, NT, rot, 0)
        for i in range(3 * NT, NCHUNK):
            b = i % 3
            gather_wait(b)
            scatter(b)
        plsc.subcore_barrier()

        @pl.when(s < _NS - 1)
        def _out_main():
            pltpu.sync_copy(acc.at[pl.ds(base, RPS)],
                            out_hbm.at[c, pl.ds(base, RPS)])

        @pl.when(s == _NS - 1)
        def _out_last():
            pltpu.sync_copy(acc.at[pl.ds(base, RLAST - _PAD)],
                            out_hbm.at[c, pl.ds(base, RLAST - _PAD)])

    return spmm


def _sc_spmm(h, gather_idx, scatter_idx):
    N = h.shape[0]
    z = jnp.zeros((N + _PAD, EMB), jnp.float32)
    return _make_sc_spmm(N, gather_idx.shape[0])(h, gather_idx, scatter_idx, z)


@functools.lru_cache(maxsize=None)
def _make_fwd(N, last):
    NB = N // BLK

    def body(p0_ref, p1_ref, h_ref, w1_ref, b1_ref, w2_ref, b2_ref,
             agg_ref, r_ref, hn_ref):
        agg = p0_ref[0] + p1_ref[0] + h_ref[...]
        agg_ref[...] = agg
        z1 = _dot(agg, w1_ref[...], 1, 0) + b1_ref[...]
        r = jnp.maximum(z1, 0.0)
        r_ref[...] = r
        z2 = _dot(r, w2_ref[...], 1, 0) + b2_ref[...]
        hn_ref[...] = z2 if last else jnp.maximum(z2, 0.0)

    blk = pl.BlockSpec((BLK, EMB), lambda i: (i, 0))
    return pl.pallas_call(
        body,
        grid=(NB,),
        in_specs=[
            pl.BlockSpec((1, BLK, EMB), lambda i: (0, i, 0)),
            pl.BlockSpec((1, BLK, EMB), lambda i: (1, i, 0)),
            blk,
            pl.BlockSpec((EMB, EMB), lambda i: (0, 0)),
            pl.BlockSpec((1, EMB), lambda i: (0, 0)),
            pl.BlockSpec((EMB, EMB), lambda i: (0, 0)),
            pl.BlockSpec((1, EMB), lambda i: (0, 0)),
        ],
        out_specs=[blk, blk, blk],
        out_shape=[jax.ShapeDtypeStruct((N, EMB), jnp.float32)] * 3,
    )


@functools.lru_cache(maxsize=None)
def _make_bwd(N, last, combine, need_dagg):
    NB = N // BLK

    def body(*refs):
        refs = list(refs)
        if combine:
            q0_ref, q1_ref, dp_ref = refs[:3]
            refs = refs[3:]
            dh = q0_ref[0] + q1_ref[0] + dp_ref[...]
        else:
            dh = refs.pop(0)[...]
        if not last:
            hn_ref = refs.pop(0)
            dh = dh * (hn_ref[...] > 0).astype(jnp.float32)
        (r_ref, agg_ref, w1_ref, b1_ref, w2_ref, b2_ref) = refs[:6]
        outs = refs[6:]
        if need_dagg:
            dagg_ref = outs.pop(0)
        w1n_ref, b1n_ref, w2n_ref, b2n_ref, aW1, ab1, aW2, ab2 = outs
        i = pl.program_id(0)

        @pl.when(i == 0)
        def _init():
            aW1[...] = jnp.zeros((EMB, EMB), jnp.float32)
            ab1[...] = jnp.zeros((1, EMB), jnp.float32)
            aW2[...] = jnp.zeros((EMB, EMB), jnp.float32)
            ab2[...] = jnp.zeros((1, EMB), jnp.float32)

        r = r_ref[...]
        aW2[...] += _dot(r, dh, 0, 0)
        ab2[...] += jnp.sum(dh, axis=0, keepdims=True)
        dr = _dot(dh, w2_ref[...], 1, 1)
        dz1 = dr * (r > 0).astype(jnp.float32)
        aW1[...] += _dot(agg_ref[...], dz1, 0, 0)
        ab1[...] += jnp.sum(dz1, axis=0, keepdims=True)
        if need_dagg:
            dagg_ref[...] = _dot(dz1, w1_ref[...], 1, 1)

        @pl.when(i == NB - 1)
        def _finish():
            w1n_ref[...] = w1_ref[...] - LR * aW1[...]
            b1n_ref[...] = b1_ref[...] - LR * ab1[...]
            w2n_ref[...] = w2_ref[...] - LR * aW2[...]
            b2n_ref[...] = b2_ref[...] - LR * ab2[...]

    blk = pl.BlockSpec((BLK, EMB), lambda i: (i, 0))
    wspec = pl.BlockSpec((EMB, EMB), lambda i: (0, 0))
    bspec = pl.BlockSpec((1, EMB), lambda i: (0, 0))
    in_specs = []
    if combine:
        in_specs += [pl.BlockSpec((1, BLK, EMB), lambda i: (0, i, 0)),
                     pl.BlockSpec((1, BLK, EMB), lambda i: (1, i, 0)),
                     blk]
    else:
        in_specs += [blk]
    if not last:
        in_specs += [blk]
    in_specs += [blk, blk, wspec, bspec, wspec, bspec]
    out_specs = []
    out_shape = []
    if need_dagg:
        out_specs += [blk]
        out_shape += [jax.ShapeDtypeStruct((N, EMB), jnp.float32)]
    out_specs += [wspec, bspec, wspec, bspec]
    out_shape += [jax.ShapeDtypeStruct((EMB, EMB), jnp.float32),
                  jax.ShapeDtypeStruct((1, EMB), jnp.float32),
                  jax.ShapeDtypeStruct((EMB, EMB), jnp.float32),
                  jax.ShapeDtypeStruct((1, EMB), jnp.float32)]
    return pl.pallas_call(
        body,
        grid=(NB,),
        in_specs=in_specs,
        out_specs=out_specs,
        out_shape=out_shape,
        scratch_shapes=[pltpu.VMEM((EMB, EMB), jnp.float32),
                        pltpu.VMEM((1, EMB), jnp.float32),
                        pltpu.VMEM((EMB, EMB), jnp.float32),
                        pltpu.VMEM((1, EMB), jnp.float32)],
    )


@functools.lru_cache(maxsize=None)
def _make_head(N):
    NB = N // BLK

    def body(h_ref, b_ref, y_ref, wgt_ref, bg_ref,
             loss_ref, ds_ref, wgtn_ref, bgn_ref, sums, cnts):
        i = pl.program_id(0)

        @pl.when(i == 0)
        def _init():
            sums[...] = jnp.zeros((EMB, EMB), jnp.float32)
            cnts[...] = jnp.zeros((EMB, EMB), jnp.float32)

        bids = b_ref[0, 0]
        lane = lax.broadcasted_iota(jnp.int32, (BLK, EMB), 1)
        oh = (lane == bids[:, None]).astype(jnp.float32)
        sums[...] += _dot(oh, h_ref[...], 0, 0)
        cnts[...] += _dot(oh, jnp.ones((BLK, EMB), jnp.float32), 0, 0)

        @pl.when(i == NB - 1)
        def _finish():
            cm = jnp.maximum(cnts[...], 1.0)
            pooled = sums[...] / cm
            wgt = wgt_ref[...]
            pred = jnp.sum(pooled * wgt, axis=1, keepdims=True) + bg_ref[...]
            y = y_ref[...]
            t = (y + 1.0) * 0.5
            valid = (y * y > 1e-5).astype(jnp.float32)
            lm = (jnp.maximum(pred, 0.0) - pred * t
                  + jnp.log1p(jnp.exp(-jnp.abs(pred))))
            vs = jnp.sum(valid)
            loss_ref[...] = jnp.reshape(jnp.sum(lm * valid) / vs, (1, 1))
            dpred = (jax.nn.sigmoid(pred) - t) * valid / vs
            ds_ref[...] = dpred * wgt / cm
            wgtn_ref[...] = wgt - LR * jnp.sum(pooled * dpred, axis=0,
                                               keepdims=True)
            bgn_ref[...] = bg_ref[...] - LR * jnp.sum(dpred)

    one = pl.BlockSpec((1, 1), lambda i: (0, 0))
    emb2 = pl.BlockSpec((EMB, EMB), lambda i: (0, 0))
    return pl.pallas_call(
        body,
        grid=(NB,),
        in_specs=[
            pl.BlockSpec((BLK, EMB), lambda i: (i, 0)),
            pl.BlockSpec((1, 1, BLK), lambda i: (i, 0, 0)),
            pl.BlockSpec((EMB, 1), lambda i: (0, 0)),
            pl.BlockSpec((1, EMB), lambda i: (0, 0)),
            one,
        ],
        out_specs=[one, emb2, pl.BlockSpec((1, EMB), lambda i: (0, 0)), one],
        out_shape=[jax.ShapeDtypeStruct((1, 1), jnp.float32),
                   jax.ShapeDtypeStruct((EMB, EMB), jnp.float32),
                   jax.ShapeDtypeStruct((1, EMB), jnp.float32),
                   jax.ShapeDtypeStruct((1, 1), jnp.float32)],
        scratch_shapes=[pltpu.VMEM((EMB, EMB), jnp.float32),
                        pltpu.VMEM((EMB, EMB), jnp.float32)],
    )


@functools.lru_cache(maxsize=None)
def _make_expand(N):
    NB = N // BLK

    def body(ds_ref, b_ref, dh_ref):
        bids = b_ref[0, 0]
        lane = lax.broadcasted_iota(jnp.int32, (BLK, EMB), 1)
        oh = (lane == bids[:, None]).astype(jnp.float32)
        dh_ref[...] = _dot(oh, ds_ref[...], 1, 0)

    return pl.pallas_call(
        body,
        grid=(NB,),
        in_specs=[
            pl.BlockSpec((EMB, EMB), lambda i: (0, 0)),
            pl.BlockSpec((1, 1, BLK), lambda i: (i, 0, 0)),
        ],
        out_specs=pl.BlockSpec((BLK, EMB), lambda i: (i, 0)),
        out_shape=jax.ShapeDtypeStruct((N, EMB), jnp.float32),
    )


def _forward(x, srcp, dstp, fw, save, P0=None):
    W1s, b1s, W2s, b2s = fw[0], fw[1], fw[2], fw[3]
    N = x.shape[0]
    h = x
    aggs, rs, hs = [], [], [h]
    for l in range(NLAYER):
        P = P0 if (l == 0 and P0 is not None) else _sc_spmm(h, srcp, dstp)
        agg, r, hn = _make_fwd(N, l == NLAYER - 1)(
            P, P, h, W1s[l], b1s[l], W2s[l], b2s[l])
        if save:
            aggs.append(agg)
            rs.append(r)
            hs.append(hn)
        h = hn
    return h, aggs, rs, hs


def kernel(x_spt, edge_index_spt, batch_spt, y_spt,
           x_qry, edge_index_qry, batch_qry, y_qry,
           W1, b1, W2, b2, Wg, bg):
    N = x_spt.shape[0]
    NB = N // BLK
    src_s = edge_index_spt[0].astype(jnp.int32)
    dst_s = edge_index_spt[1].astype(jnp.int32)
    src_q = edge_index_qry[0].astype(jnp.int32)
    dst_q = edge_index_qry[1].astype(jnp.int32)
    sf_g, sf_s = src_s, dst_s
    st_g, st_s = dst_s, src_s
    qf_g, qf_s = src_q, dst_q
    batch_s3 = batch_spt.astype(jnp.int32).reshape(NB, 1, BLK)
    batch_q3 = batch_qry.astype(jnp.int32).reshape(NB, 1, BLK)
    y_s = jnp.pad(y_spt, (0, EMB - y_spt.shape[0])).reshape(EMB, 1)
    y_q = jnp.pad(y_qry, (0, EMB - y_qry.shape[0])).reshape(EMB, 1)

    fW1 = [W1[l] for l in range(NLAYER)]
    fb1 = [b1[l].reshape(1, EMB) for l in range(NLAYER)]
    fW2 = [W2[l] for l in range(NLAYER)]
    fb2 = [b2[l].reshape(1, EMB) for l in range(NLAYER)]
    fwgT = Wg.reshape(1, EMB)
    fbg = bg.reshape(1, 1)

    P0_spt = _sc_spmm(x_spt, sf_g, sf_s)
    for _ in range(NSTEP):
        h, aggs, rs, hs = _forward(x_spt, sf_g, sf_s,
                                   (fW1, fb1, fW2, fb2), save=True, P0=P0_spt)
        _, d_sums, fwgT_new, fbg_new = _make_head(N)(h, batch_s3, y_s, fwgT, fbg)
        dh = _make_expand(N)(d_sums, batch_s3)
        nW1 = [None] * NLAYER
        nb1 = [None] * NLAYER
        nW2 = [None] * NLAYER
        nb2 = [None] * NLAYER
        dprev = None
        Q = None
        for l in range(NLAYER - 1, -1, -1):
            last = l == NLAYER - 1
            need_dagg = l > 0
            bwd = _make_bwd(N, last, not last, need_dagg)
            args = []
            if last:
                args += [dh]
            else:
                args += [Q, Q, dprev, hs[l + 1]]
            args += [rs[l], aggs[l], fW1[l], fb1[l], fW2[l], fb2[l]]
            outs = bwd(*args)
            if need_dagg:
                dagg = outs[0]
                outs = outs[1:]
                Q = _sc_spmm(dagg, st_g, st_s)
                dprev = dagg
            nW1[l], nb1[l], nW2[l], nb2[l] = outs
        fW1, fb1, fW2, fb2 = nW1, nb1, nW2, nb2
        fwgT, fbg = fwgT_new, fbg_new

    h, _, _, _ = _forward(x_qry, qf_g, qf_s,
                          (fW1, fb1, fW2, fb2), save=False)
    loss, _, _, _ = _make_head(N)(h, batch_q3, y_q, fwgT, fbg)
    return loss[0, 0]


# speedup vs baseline: 3.4439x; 1.1185x over previous
import functools

import jax
import jax.numpy as jnp
from jax import lax
from jax.experimental import pallas as pl
from jax.experimental.pallas import tpu as pltpu
from jax.experimental.pallas import tpu_sc as plsc

EMB = 128
NLAYER = 5
LR = 0.01
NSTEP = 3
BLK = 1000

_HI = lax.Precision.HIGHEST


def _dot(a, b, ca, cb):
    return lax.dot_general(a, b, (((ca,), (cb,)), ((), ())),
                           precision=_HI, preferred_element_type=jnp.float32)


_NC, _NS = 2, 16
_NW = _NC * _NS
_PAD = 16


@functools.lru_cache(maxsize=None)
def _make_sc_spmm(N, E):
    EW = E // _NW
    KE = 80
    NCHUNK = EW // KE
    RPS = (N // _NS) & ~7
    RLAST = N + _PAD - (_NS - 1) * RPS
    mesh = plsc.VectorSubcoreMesh(core_axis_name="c", subcore_axis_name="s")

    @functools.partial(
        pl.kernel,
        mesh=mesh,
        out_type=jax.ShapeDtypeStruct((_NC, N, EMB), jnp.float32),
        scratch_types=[
            pltpu.VMEM((6, KE), jnp.int32),
            pltpu.VMEM((6, KE), jnp.int32),
            pltpu.VMEM((3, KE, EMB), jnp.float32),
            pltpu.VMEM_SHARED((N + _PAD, EMB), jnp.float32),
            [pltpu.SemaphoreType.DMA] * 3,
            [pltpu.SemaphoreType.DMA] * 3,
            [pltpu.SemaphoreType.DMA] * 12,
        ],
    )
    def spmm(h_hbm, src_hbm, dst_hbm, z_hbm, out_hbm,
             src_v, dst_v, rows_v, acc, gsems, ssems, isems):
        c = lax.axis_index("c")
        s = lax.axis_index("s")
        wid = s * _NC + c
        base = s * RPS

        @pl.when(s < _NS - 1)
        def _zero_main():
            pltpu.sync_copy(z_hbm.at[pl.ds(base, RPS)],
                            acc.at[pl.ds(base, RPS)])

        @pl.when(s == _NS - 1)
        def _zero_last():
            pltpu.sync_copy(z_hbm.at[pl.ds(base, RLAST)],
                            acc.at[pl.ds(base, RLAST)])

        plsc.subcore_barrier()

        def idx_start(q, i):
            e0 = wid * EW + i * KE
            pltpu.async_copy(src_hbm.at[pl.ds(e0, KE)], src_v.at[q],
                             isems[2 * q])
            pltpu.async_copy(dst_hbm.at[pl.ds(e0, KE)], dst_v.at[q],
                             isems[2 * q + 1])

        def idx_wait(q, i):
            e0 = wid * EW + i * KE
            pltpu.make_async_copy(src_hbm.at[pl.ds(e0, KE)], src_v.at[q],
                                  isems[2 * q]).wait()
            pltpu.make_async_copy(dst_hbm.at[pl.ds(e0, KE)], dst_v.at[q],
                                  isems[2 * q + 1]).wait()

        def gather_start(b, q):
            pltpu.async_copy(h_hbm.at[src_v.at[q]], rows_v.at[b], gsems[b])

        def gather_wait(b, q):
            pltpu.make_async_copy(h_hbm.at[src_v.at[q]], rows_v.at[b],
                                  gsems[b]).wait()

        def scatter_start(b, q):
            pltpu.async_copy(rows_v.at[b], acc.at[dst_v.at[q]], ssems[b],
                             add=True)

        def scatter_wait(b, q):
            pltpu.make_async_copy(rows_v.at[b], acc.at[dst_v.at[q]],
                                  ssems[b]).wait()

        for j in range(5):
            idx_start(j % 6, j)
        idx_wait(0, 0)
        gather_start(0, 0)
        idx_wait(1, 1)
        gather_start(1, 1)

        def chunk_body(i, b, q, first):
            gather_wait(b, q)
            scatter_start(b, q)
            if first:
                @pl.when(i >= 1)
                def _wprev():
                    scatter_wait((b + 2) % 3, (q + 5) % 6)
            else:
                scatter_wait((b + 2) % 3, (q + 5) % 6)

            @pl.when(i + 2 < NCHUNK)
            def _regather():
                idx_wait((q + 2) % 6, i + 2)
                gather_start((b + 2) % 3, (q + 2) % 6)

            @pl.when(i + 5 < NCHUNK)
            def _refill():
                idx_start((q + 5) % 6, i + 5)

        NT = NCHUNK // 6

        def rot(p, carry):
            i0 = 6 * p
            for j in range(6):
                chunk_body(i0 + j, j % 3, j, first=(j == 0))
            return carry

        lax.fori_loop(0, NT, rot, 0)
        for i in range(6 * NT, NCHUNK):
            chunk_body(jnp.int32(i), i % 3, i % 6, first=False)
        scatter_wait((NCHUNK - 1) % 3, (NCHUNK - 1) % 6)
        plsc.subcore_barrier()

        @pl.when(s < _NS - 1)
        def _out_main():
            pltpu.sync_copy(acc.at[pl.ds(base, RPS)],
                            out_hbm.at[c, pl.ds(base, RPS)])

        @pl.when(s == _NS - 1)
        def _out_last():
            pltpu.sync_copy(acc.at[pl.ds(base, RLAST - _PAD)],
                            out_hbm.at[c, pl.ds(base, RLAST - _PAD)])

    return spmm


def _sc_spmm(h, gather_idx, scatter_idx):
    N = h.shape[0]
    z = jnp.zeros((N + _PAD, EMB), jnp.float32)
    return _make_sc_spmm(N, gather_idx.shape[0])(h, gather_idx, scatter_idx, z)


@functools.lru_cache(maxsize=None)
def _make_fwd(N, last):
    NB = N // BLK

    def body(p0_ref, p1_ref, h_ref, w1_ref, b1_ref, w2_ref, b2_ref,
             agg_ref, r_ref, hn_ref):
        agg = p0_ref[0] + p1_ref[0] + h_ref[...]
        agg_ref[...] = agg
        z1 = _dot(agg, w1_ref[...], 1, 0) + b1_ref[...]
        r = jnp.maximum(z1, 0.0)
        r_ref[...] = r
        z2 = _dot(r, w2_ref[...], 1, 0) + b2_ref[...]
        hn_ref[...] = z2 if last else jnp.maximum(z2, 0.0)

    blk = pl.BlockSpec((BLK, EMB), lambda i: (i, 0))
    return pl.pallas_call(
        body,
        grid=(NB,),
        in_specs=[
            pl.BlockSpec((1, BLK, EMB), lambda i: (0, i, 0)),
            pl.BlockSpec((1, BLK, EMB), lambda i: (1, i, 0)),
            blk,
            pl.BlockSpec((EMB, EMB), lambda i: (0, 0)),
            pl.BlockSpec((1, EMB), lambda i: (0, 0)),
            pl.BlockSpec((EMB, EMB), lambda i: (0, 0)),
            pl.BlockSpec((1, EMB), lambda i: (0, 0)),
        ],
        out_specs=[blk, blk, blk],
        out_shape=[jax.ShapeDtypeStruct((N, EMB), jnp.float32)] * 3,
    )


@functools.lru_cache(maxsize=None)
def _make_bwd(N, last, combine, need_dagg):
    NB = N // BLK

    def body(*refs):
        refs = list(refs)
        if combine:
            q0_ref, q1_ref, dp_ref = refs[:3]
            refs = refs[3:]
            dh = q0_ref[0] + q1_ref[0] + dp_ref[...]
        else:
            dh = refs.pop(0)[...]
        if not last:
            hn_ref = refs.pop(0)
            dh = dh * (hn_ref[...] > 0).astype(jnp.float32)
        (r_ref, agg_ref, w1_ref, b1_ref, w2_ref, b2_ref) = refs[:6]
        outs = refs[6:]
        if need_dagg:
            dagg_ref = outs.pop(0)
        w1n_ref, b1n_ref, w2n_ref, b2n_ref, aW1, ab1, aW2, ab2 = outs
        i = pl.program_id(0)

        @pl.when(i == 0)
        def _init():
            aW1[...] = jnp.zeros((EMB, EMB), jnp.float32)
            ab1[...] = jnp.zeros((1, EMB), jnp.float32)
            aW2[...] = jnp.zeros((EMB, EMB), jnp.float32)
            ab2[...] = jnp.zeros((1, EMB), jnp.float32)

        r = r_ref[...]
        aW2[...] += _dot(r, dh, 0, 0)
        ab2[...] += jnp.sum(dh, axis=0, keepdims=True)
        dr = _dot(dh, w2_ref[...], 1, 1)
        dz1 = dr * (r > 0).astype(jnp.float32)
        aW1[...] += _dot(agg_ref[...], dz1, 0, 0)
        ab1[...] += jnp.sum(dz1, axis=0, keepdims=True)
        if need_dagg:
            dagg_ref[...] = _dot(dz1, w1_ref[...], 1, 1)

        @pl.when(i == NB - 1)
        def _finish():
            w1n_ref[...] = w1_ref[...] - LR * aW1[...]
            b1n_ref[...] = b1_ref[...] - LR * ab1[...]
            w2n_ref[...] = w2_ref[...] - LR * aW2[...]
            b2n_ref[...] = b2_ref[...] - LR * ab2[...]

    blk = pl.BlockSpec((BLK, EMB), lambda i: (i, 0))
    wspec = pl.BlockSpec((EMB, EMB), lambda i: (0, 0))
    bspec = pl.BlockSpec((1, EMB), lambda i: (0, 0))
    in_specs = []
    if combine:
        in_specs += [pl.BlockSpec((1, BLK, EMB), lambda i: (0, i, 0)),
                     pl.BlockSpec((1, BLK, EMB), lambda i: (1, i, 0)),
                     blk]
    else:
        in_specs += [blk]
    if not last:
        in_specs += [blk]
    in_specs += [blk, blk, wspec, bspec, wspec, bspec]
    out_specs = []
    out_shape = []
    if need_dagg:
        out_specs += [blk]
        out_shape += [jax.ShapeDtypeStruct((N, EMB), jnp.float32)]
    out_specs += [wspec, bspec, wspec, bspec]
    out_shape += [jax.ShapeDtypeStruct((EMB, EMB), jnp.float32),
                  jax.ShapeDtypeStruct((1, EMB), jnp.float32),
                  jax.ShapeDtypeStruct((EMB, EMB), jnp.float32),
                  jax.ShapeDtypeStruct((1, EMB), jnp.float32)]
    return pl.pallas_call(
        body,
        grid=(NB,),
        in_specs=in_specs,
        out_specs=out_specs,
        out_shape=out_shape,
        scratch_shapes=[pltpu.VMEM((EMB, EMB), jnp.float32),
                        pltpu.VMEM((1, EMB), jnp.float32),
                        pltpu.VMEM((EMB, EMB), jnp.float32),
                        pltpu.VMEM((1, EMB), jnp.float32)],
    )


@functools.lru_cache(maxsize=None)
def _make_head(N):
    NB = N // BLK

    def body(h_ref, b_ref, y_ref, wgt_ref, bg_ref,
             loss_ref, ds_ref, wgtn_ref, bgn_ref, sums, cnts):
        i = pl.program_id(0)

        @pl.when(i == 0)
        def _init():
            sums[...] = jnp.zeros((EMB, EMB), jnp.float32)
            cnts[...] = jnp.zeros((EMB, EMB), jnp.float32)

        bids = b_ref[0, 0]
        lane = lax.broadcasted_iota(jnp.int32, (BLK, EMB), 1)
        oh = (lane == bids[:, None]).astype(jnp.float32)
        sums[...] += _dot(oh, h_ref[...], 0, 0)
        cnts[...] += _dot(oh, jnp.ones((BLK, EMB), jnp.float32), 0, 0)

        @pl.when(i == NB - 1)
        def _finish():
            cm = jnp.maximum(cnts[...], 1.0)
            pooled = sums[...] / cm
            wgt = wgt_ref[...]
            pred = jnp.sum(pooled * wgt, axis=1, keepdims=True) + bg_ref[...]
            y = y_ref[...]
            t = (y + 1.0) * 0.5
            valid = (y * y > 1e-5).astype(jnp.float32)
            lm = (jnp.maximum(pred, 0.0) - pred * t
                  + jnp.log1p(jnp.exp(-jnp.abs(pred))))
            vs = jnp.sum(valid)
            loss_ref[...] = jnp.reshape(jnp.sum(lm * valid) / vs, (1, 1))
            dpred = (jax.nn.sigmoid(pred) - t) * valid / vs
            ds_ref[...] = dpred * wgt / cm
            wgtn_ref[...] = wgt - LR * jnp.sum(pooled * dpred, axis=0,
                                               keepdims=True)
            bgn_ref[...] = bg_ref[...] - LR * jnp.sum(dpred)

    one = pl.BlockSpec((1, 1), lambda i: (0, 0))
    emb2 = pl.BlockSpec((EMB, EMB), lambda i: (0, 0))
    return pl.pallas_call(
        body,
        grid=(NB,),
        in_specs=[
            pl.BlockSpec((BLK, EMB), lambda i: (i, 0)),
            pl.BlockSpec((1, 1, BLK), lambda i: (i, 0, 0)),
            pl.BlockSpec((EMB, 1), lambda i: (0, 0)),
            pl.BlockSpec((1, EMB), lambda i: (0, 0)),
            one,
        ],
        out_specs=[one, emb2, pl.BlockSpec((1, EMB), lambda i: (0, 0)), one],
        out_shape=[jax.ShapeDtypeStruct((1, 1), jnp.float32),
                   jax.ShapeDtypeStruct((EMB, EMB), jnp.float32),
                   jax.ShapeDtypeStruct((1, EMB), jnp.float32),
                   jax.ShapeDtypeStruct((1, 1), jnp.float32)],
        scratch_shapes=[pltpu.VMEM((EMB, EMB), jnp.float32),
                        pltpu.VMEM((EMB, EMB), jnp.float32)],
    )


@functools.lru_cache(maxsize=None)
def _make_expand(N):
    NB = N // BLK

    def body(ds_ref, b_ref, dh_ref):
        bids = b_ref[0, 0]
        lane = lax.broadcasted_iota(jnp.int32, (BLK, EMB), 1)
        oh = (lane == bids[:, None]).astype(jnp.float32)
        dh_ref[...] = _dot(oh, ds_ref[...], 1, 0)

    return pl.pallas_call(
        body,
        grid=(NB,),
        in_specs=[
            pl.BlockSpec((EMB, EMB), lambda i: (0, 0)),
            pl.BlockSpec((1, 1, BLK), lambda i: (i, 0, 0)),
        ],
        out_specs=pl.BlockSpec((BLK, EMB), lambda i: (i, 0)),
        out_shape=jax.ShapeDtypeStruct((N, EMB), jnp.float32),
    )


def _forward(x, srcp, dstp, fw, save, P0=None):
    W1s, b1s, W2s, b2s = fw[0], fw[1], fw[2], fw[3]
    N = x.shape[0]
    h = x
    aggs, rs, hs = [], [], [h]
    for l in range(NLAYER):
        P = P0 if (l == 0 and P0 is not None) else _sc_spmm(h, srcp, dstp)
        agg, r, hn = _make_fwd(N, l == NLAYER - 1)(
            P, P, h, W1s[l], b1s[l], W2s[l], b2s[l])
        if save:
            aggs.append(agg)
            rs.append(r)
            hs.append(hn)
        h = hn
    return h, aggs, rs, hs


def kernel(x_spt, edge_index_spt, batch_spt, y_spt,
           x_qry, edge_index_qry, batch_qry, y_qry,
           W1, b1, W2, b2, Wg, bg):
    N = x_spt.shape[0]
    NB = N // BLK
    src_s = edge_index_spt[0].astype(jnp.int32)
    dst_s = edge_index_spt[1].astype(jnp.int32)
    src_q = edge_index_qry[0].astype(jnp.int32)
    dst_q = edge_index_qry[1].astype(jnp.int32)
    sf_g, sf_s = src_s, dst_s
    st_g, st_s = dst_s, src_s
    qf_g, qf_s = src_q, dst_q
    batch_s3 = batch_spt.astype(jnp.int32).reshape(NB, 1, BLK)
    batch_q3 = batch_qry.astype(jnp.int32).reshape(NB, 1, BLK)
    y_s = jnp.pad(y_spt, (0, EMB - y_spt.shape[0])).reshape(EMB, 1)
    y_q = jnp.pad(y_qry, (0, EMB - y_qry.shape[0])).reshape(EMB, 1)

    fW1 = [W1[l] for l in range(NLAYER)]
    fb1 = [b1[l].reshape(1, EMB) for l in range(NLAYER)]
    fW2 = [W2[l] for l in range(NLAYER)]
    fb2 = [b2[l].reshape(1, EMB) for l in range(NLAYER)]
    fwgT = Wg.reshape(1, EMB)
    fbg = bg.reshape(1, 1)

    P0_spt = _sc_spmm(x_spt, sf_g, sf_s)
    for _ in range(NSTEP):
        h, aggs, rs, hs = _forward(x_spt, sf_g, sf_s,
                                   (fW1, fb1, fW2, fb2), save=True, P0=P0_spt)
        _, d_sums, fwgT_new, fbg_new = _make_head(N)(h, batch_s3, y_s, fwgT, fbg)
        dh = _make_expand(N)(d_sums, batch_s3)
        nW1 = [None] * NLAYER
        nb1 = [None] * NLAYER
        nW2 = [None] * NLAYER
        nb2 = [None] * NLAYER
        dprev = None
        Q = None
        for l in range(NLAYER - 1, -1, -1):
            last = l == NLAYER - 1
            need_dagg = l > 0
            bwd = _make_bwd(N, last, not last, need_dagg)
            args = []
            if last:
                args += [dh]
            else:
                args += [Q, Q, dprev, hs[l + 1]]
            args += [rs[l], aggs[l], fW1[l], fb1[l], fW2[l], fb2[l]]
            outs = bwd(*args)
            if need_dagg:
                dagg = outs[0]
                outs = outs[1:]
                Q = _sc_spmm(dagg, st_g, st_s)
                dprev = dagg
            nW1[l], nb1[l], nW2[l], nb2[l] = outs
        fW1, fb1, fW2, fb2 = nW1, nb1, nW2, nb2
        fwgT, fbg = fwgT_new, fbg_new

    h, _, _, _ = _forward(x_qry, qf_g, qf_s,
                          (fW1, fb1, fW2, fb2), save=False)
    loss, _, _, _ = _make_head(N)(h, batch_q3, y_q, fwgT, fbg)
    return loss[0, 0]
